# Initial kernel scaffold; baseline (speedup 1.0000x reference)
import jax
import jax.numpy as jnp
from jax.experimental import pallas as pl


def kernel(x, edge_index, edge_attr, batch, params):
    raise NotImplementedError("write your pallas kernel here")


# speedup vs baseline: 2.7706x; 2.7706x over previous
import functools

import jax
import jax.numpy as jnp
import numpy as np
from jax import lax
from jax.experimental import pallas as pl
from jax.experimental.pallas import tpu as pltpu
from jax.experimental.pallas import tpu_sc as plsc

N_NODES = 4096
N_EDGES = 49152
N_GRAPHS = 128
CONV_LAYERS = 5
ATOM_F = 92
RBF_BINS = 128
C = 128
H = 4
HC = H * C

F32 = jnp.float32

SC_CORES = 2
SC_SUBCORES = 16
SC_WORKERS = SC_CORES * SC_SUBCORES

_SC_MESH = dict(core_axis_name="c", subcore_axis_name="s")


def _dot(a, b):
    return jnp.dot(a, b, preferred_element_type=F32,
                   precision=lax.Precision.HIGHEST)


def _prep_body(x_ref, aW_ref, ab_ref, o_ref):
    o_ref[...] = _dot(x_ref[...], aW_ref[...]) + ab_ref[...]


def _prep_call(x, aW, ab):
    return pl.pallas_call(
        _prep_body,
        out_shape=jax.ShapeDtypeStruct((N_NODES, C), F32),
    )(x, aW, ab.reshape(1, C))


def _ef_body(d2_ref, W1_ref, b1_ref, W2_ref, b2_ref, o_ref):
    d = jnp.sqrt(d2_ref[...])
    centers = lax.broadcasted_iota(jnp.int32, (1, RBF_BINS), 1).astype(F32) * (
        8.0 / (RBF_BINS - 1))
    gamma = 1.0 / (8.0 / (RBF_BINS - 1))
    rbf = jnp.exp(-gamma * (d - centers) ** 2)
    h = _dot(rbf, W1_ref[...]) + b1_ref[...]
    sp = jnp.maximum(h, 0.0) + jnp.log1p(jnp.exp(-jnp.abs(h)))
    o_ref[...] = _dot(sp, W2_ref[...]) + b2_ref[...]


def _ef_call(d2, W1, b1, W2, b2):
    B = 512
    grid = (N_EDGES // B,)
    return pl.pallas_call(
        _ef_body,
        grid=grid,
        in_specs=[
            pl.BlockSpec((B, 1), lambda i: (i, 0)),
            pl.BlockSpec((RBF_BINS, C), lambda i: (0, 0)),
            pl.BlockSpec((1, C), lambda i: (0, 0)),
            pl.BlockSpec((C, C), lambda i: (0, 0)),
            pl.BlockSpec((1, C), lambda i: (0, 0)),
        ],
        out_specs=pl.BlockSpec((B, C), lambda i: (i, 0)),
        out_shape=jax.ShapeDtypeStruct((N_EDGES, C), F32),
    )(d2, W1, b1.reshape(1, C), W2, b2.reshape(1, C))


def _qkv_body(nf_ref, Wq_ref, bq_ref, Wk_ref, bk_ref, Wv_ref, bv_ref,
              dst_ref, src_ref):
    nf = nf_ref[...]
    q = _dot(nf, Wq_ref[...]) + bq_ref[...]
    k = _dot(nf, Wk_ref[...]) + bk_ref[...]
    v = _dot(nf, Wv_ref[...]) + bv_ref[...]
    dst_ref[:, 0:HC] = q
    dst_ref[:, HC:2 * HC] = q * k
    dst_ref[:, 2 * HC:3 * HC] = v
    src_ref[:, 0:HC] = k
    src_ref[:, HC:2 * HC] = v


def _qkv_call(nf, Wq, bq, Wk, bk, Wv, bv):
    B = 1024
    grid = (N_NODES // B,)
    wspec = pl.BlockSpec((C, HC), lambda i: (0, 0))
    bspec = pl.BlockSpec((1, HC), lambda i: (0, 0))
    return pl.pallas_call(
        _qkv_body,
        grid=grid,
        in_specs=[pl.BlockSpec((B, C), lambda i: (i, 0)),
                  wspec, bspec, wspec, bspec, wspec, bspec],
        out_specs=(pl.BlockSpec((B, 3 * HC), lambda i: (i, 0)),
                   pl.BlockSpec((B, 2 * HC), lambda i: (i, 0))),
        out_shape=(jax.ShapeDtypeStruct((N_NODES, 3 * HC), F32),
                   jax.ShapeDtypeStruct((N_NODES, 2 * HC), F32)),
    )(nf, Wq, bq.reshape(1, HC), Wk, bk.reshape(1, HC), Wv, bv.reshape(1, HC))


def _edge_body(gdst_ref, gsrc_ref, ef_ref,
               We_ref, be_ref, Wm_ref, bm_ref, Wmsg_ref, bmsg_ref,
               lng_ref, lnb_ref, lnmg_ref, lnmb_ref, o_ref):
    scale = 1.0 / np.sqrt(3.0 * C)
    e_full = _dot(ef_ref[...], We_ref[...]) + be_ref[...]
    Wm = Wm_ref[...]
    Wmsg = Wmsg_ref[...]
    bm = bm_ref[...]
    lng = lng_ref[...]
    lnb = lnb_ref[...]
    for h in range(H):
        sl = slice(h * C, (h + 1) * C)
        q_h = gdst_ref[:, h * C:(h + 1) * C]
        g1_h = gdst_ref[:, HC + h * C:HC + (h + 1) * C]
        vd_h = gdst_ref[:, 2 * HC + h * C:2 * HC + (h + 1) * C]
        ks_h = gsrc_ref[:, h * C:(h + 1) * C]
        vs_h = gsrc_ref[:, HC + h * C:HC + (h + 1) * C]
        e_h = e_full[:, sl]
        a1 = g1_h * scale
        a2 = (q_h * ks_h) * scale
        a3 = (q_h * e_h) * scale
        m = (jnp.sum(a1, 1, keepdims=True) + jnp.sum(a2, 1, keepdims=True)
             + jnp.sum(a3, 1, keepdims=True)) * (1.0 / (3 * C))
        d1 = a1 - m
        d2 = a2 - m
        d3 = a3 - m
        var = (jnp.sum(d1 * d1, 1, keepdims=True)
               + jnp.sum(d2 * d2, 1, keepdims=True)
               + jnp.sum(d3 * d3, 1, keepdims=True)) * (1.0 / (3 * C))
        rstd = lax.rsqrt(var + 1e-5)
        s1 = jax.nn.sigmoid(d1 * rstd * lng[:, 0:C] + lnb[:, 0:C])
        s2 = jax.nn.sigmoid(d2 * rstd * lng[:, C:2 * C] + lnb[:, C:2 * C])
        s3 = jax.nn.sigmoid(d3 * rstd * lng[:, 2 * C:3 * C] + lnb[:, 2 * C:3 * C])
        sig = jnp.concatenate([s1, s2, s3], axis=1)
        msg = (_dot(vd_h, Wm[0:C, :]) + _dot(vs_h, Wm[C:2 * C, :])
               + _dot(e_h, Wm[2 * C:3 * C, :]) + bm)
        msg = msg * sig
        msg2 = _dot(msg, Wmsg) + bmsg_ref[...]
        m2 = jnp.mean(msg2, 1, keepdims=True)
        dv = msg2 - m2
        v2 = jnp.mean(dv * dv, 1, keepdims=True)
        res = dv * lax.rsqrt(v2 + 1e-5) * lnmg_ref[...] + lnmb_ref[...]
        o_ref[sl, :] = res.T


def _edge_call(gdst, gsrc, ef, We, be, Wm, bm, Wmsg, bmsg, lng, lnb, lnmg, lnmb):
    B = 512
    grid = (N_EDGES // B,)
    return pl.pallas_call(
        _edge_body,
        grid=grid,
        in_specs=[
            pl.BlockSpec((B, 3 * HC), lambda i: (i, 0)),
            pl.BlockSpec((B, 2 * HC), lambda i: (i, 0)),
            pl.BlockSpec((B, C), lambda i: (i, 0)),
            pl.BlockSpec((C, HC), lambda i: (0, 0)),
            pl.BlockSpec((1, HC), lambda i: (0, 0)),
            pl.BlockSpec((3 * C, 3 * C), lambda i: (0, 0)),
            pl.BlockSpec((1, 3 * C), lambda i: (0, 0)),
            pl.BlockSpec((3 * C, C), lambda i: (0, 0)),
            pl.BlockSpec((1, C), lambda i: (0, 0)),
            pl.BlockSpec((1, 3 * C), lambda i: (0, 0)),
            pl.BlockSpec((1, 3 * C), lambda i: (0, 0)),
            pl.BlockSpec((1, C), lambda i: (0, 0)),
            pl.BlockSpec((1, C), lambda i: (0, 0)),
        ],
        out_specs=pl.BlockSpec((HC, B), lambda i: (0, i)),
        out_shape=jax.ShapeDtypeStruct((HC, N_EDGES), F32),
    )(gdst, gsrc, ef, We, be.reshape(1, HC), Wm, bm.reshape(1, 3 * C),
      Wmsg, bmsg.reshape(1, C), lng.reshape(1, 3 * C), lnb.reshape(1, 3 * C),
      lnmg.reshape(1, C), lnmb.reshape(1, C))


def _epi_body(agg_ref, nf_ref, Wc_ref, bc_ref, bng_ref, bnb_ref,
              Ws_ref, bs_ref, o_ref):
    out = lax.dot_general(agg_ref[...], Wc_ref[...],
                          (((0,), (0,)), ((), ())),
                          preferred_element_type=F32,
                          precision=lax.Precision.HIGHEST) + bc_ref[...]
    mu = jnp.mean(out, 0, keepdims=True)
    dv = out - mu
    var = jnp.mean(dv * dv, 0, keepdims=True)
    out = dv * lax.rsqrt(var + 1e-5) * bng_ref[...] + bnb_ref[...]
    out = out * jax.nn.sigmoid(out)
    o_ref[...] = out + _dot(nf_ref[...], Ws_ref[...]) + bs_ref[...]


def _epi_call(agg, nf, Wc, bc, bng, bnb, Ws, bs):
    return pl.pallas_call(
        _epi_body,
        out_shape=jax.ShapeDtypeStruct((N_NODES, C), F32),
    )(agg, nf, Wc, bc.reshape(1, C), bng.reshape(1, C), bnb.reshape(1, C),
      Ws, bs.reshape(1, C))


def _pool_body(nf_ref, batch_ref, fcW_ref, fcb_ref, oW_ref, ob_ref, o_ref):
    ids = lax.broadcasted_iota(jnp.int32, (N_GRAPHS, 1), 0)
    oh = (batch_ref[...] == ids).astype(F32)
    counts = jnp.sum(oh, 1, keepdims=True)
    pooled = _dot(oh, nf_ref[...]) / jnp.maximum(counts, 1.0)
    feat = _dot(pooled, fcW_ref[...]) + fcb_ref[...]
    feat = feat * jax.nn.sigmoid(feat)
    o_ref[...] = _dot(feat, oW_ref[...]) + ob_ref[...]


def _pool_call(nf, batch_row, fcW, fcb, oW, ob):
    return pl.pallas_call(
        _pool_body,
        out_shape=jax.ShapeDtypeStruct((N_GRAPHS, 1), F32),
    )(nf, batch_row, fcW, fcb.reshape(1, C), oW, ob.reshape(1, 1))


def _sc_gather(table, idx, width, gch):
    epw = N_EDGES // SC_WORKERS

    @functools.partial(
        pl.kernel,
        mesh=plsc.VectorSubcoreMesh(**_SC_MESH),
        out_type=jax.ShapeDtypeStruct((N_EDGES, width), F32),
        scratch_types=[
            pltpu.VMEM((gch,), jnp.int32),
            pltpu.VMEM((gch, width), F32),
            pltpu.SemaphoreType.DMA,
        ],
    )
    def k(table_hbm, idx_hbm, out_hbm, idx_v, rows_v, sem):
        wid = lax.axis_index("s") * SC_CORES + lax.axis_index("c")
        base = wid * epw

        def body(i, carry):
            off = base + i * gch
            pltpu.sync_copy(idx_hbm.at[pl.ds(off, gch)], idx_v)
            pltpu.async_copy(table_hbm.at[idx_v], rows_v, sem).wait()
            pltpu.sync_copy(rows_v, out_hbm.at[pl.ds(off, gch)])
            return carry

        lax.fori_loop(0, epw // gch, body, 0)

    return k(table, idx)


def _sc_scatter_add(msg2_t, idx, zeros):
    rows = 16
    sch = 128

    @functools.partial(
        pl.kernel,
        mesh=plsc.VectorSubcoreMesh(**_SC_MESH),
        out_type=jax.ShapeDtypeStruct((HC, N_NODES), F32),
        scratch_types=[
            pltpu.VMEM((sch,), jnp.int32),
            pltpu.VMEM((rows, sch), F32),
            pltpu.VMEM((rows, N_NODES), F32),
            pltpu.SemaphoreType.DMA,
        ],
        compiler_params=pltpu.CompilerParams(needs_layout_passes=False),
    )
    def k(msg_hbm, idx_hbm, zeros_hbm, agg_hbm, idx_v, rows_v, acc_v, sem):
        wid = lax.axis_index("s") * SC_CORES + lax.axis_index("c")
        r0 = wid * rows
        pltpu.sync_copy(zeros_hbm, acc_v)
        lanes = lax.iota(jnp.int32, 16)

        def body(i, carry):
            off = i * sch
            pltpu.sync_copy(idx_hbm.at[pl.ds(off, sch)], idx_v)
            pltpu.sync_copy(msg_hbm.at[pl.ds(r0, rows), pl.ds(off, sch)],
                            rows_v)

            def group(g, carry2):
                dstv = idx_v[pl.ds(g * 16, 16)]
                for j in range(16):
                    jv = jnp.full((16,), j, jnp.int32)
                    vals = plsc.load_gather(rows_v, [lanes, jnp.full((16,), g * 16 + j, jnp.int32)])
                    dstb = jnp.take(dstv, jv, mode="wrap")
                    plsc.addupdate_scatter(acc_v, [lanes, dstb], vals)
                return carry2

            lax.fori_loop(0, sch // 16, group, 0)
            return carry

        lax.fori_loop(0, N_EDGES // sch, body, 0)
        pltpu.sync_copy(acc_v, agg_hbm.at[pl.ds(r0, rows)])

    return k(msg2_t, idx, zeros)


def kernel(x, edge_index, edge_attr, batch, params):
    p = params
    src = edge_index[0].astype(jnp.int32)
    dst = edge_index[1].astype(jnp.int32)
    batch_row = batch.astype(jnp.int32).reshape(1, N_NODES)
    d2 = jnp.sum(edge_attr * edge_attr, axis=1, keepdims=True)
    zeros = jnp.zeros((16, N_NODES), F32)

    nf = _prep_call(x, p['atom_W'], p['atom_b'])
    ef = _ef_call(d2, p['rbf_W1'], p['rbf_b1'], p['rbf_W2'], p['rbf_b2'])

    for l in range(CONV_LAYERS):
        tdst, tsrc = _qkv_call(nf, p['Wq'][l], p['bq'][l], p['Wk'][l],
                               p['bk'][l], p['Wv'][l], p['bv'][l])
        gdst = _sc_gather(tdst, dst, 3 * HC, 64)
        gsrc = _sc_gather(tsrc, src, 2 * HC, 96)
        msg2 = _edge_call(gdst, gsrc, ef, p['We'][l], p['be'][l], p['Wm'][l],
                          p['bm'][l], p['Wmsg'][l], p['bmsg'][l],
                          p['ln_g'][l], p['ln_b'][l], p['lnm_g'][l],
                          p['lnm_b'][l])
        agg = _sc_scatter_add(msg2, dst, zeros)
        nf = _epi_call(agg, nf, p['Wc'][l], p['bc'][l], p['bn_g'][l],
                       p['bn_b'][l], p['Ws'][l], p['bs'][l])

    out = _pool_call(nf, batch_row, p['fc_W'], p['fc_b'], p['out_W'], p['out_b'])
    return out.reshape(N_GRAPHS)


# speedup vs baseline: 3.2177x; 1.1614x over previous
import functools

import jax
import jax.numpy as jnp
import numpy as np
from jax import lax
from jax.experimental import pallas as pl
from jax.experimental.pallas import tpu as pltpu
from jax.experimental.pallas import tpu_sc as plsc

N_NODES = 4096
N_EDGES = 49152
N_GRAPHS = 128
CONV_LAYERS = 5
ATOM_F = 92
RBF_BINS = 128
C = 128
H = 4
HC = H * C

F32 = jnp.float32

SC_CORES = 2
SC_SUBCORES = 16
SC_WORKERS = SC_CORES * SC_SUBCORES

_SC_MESH = dict(core_axis_name="c", subcore_axis_name="s")


def _dot(a, b):
    return jnp.dot(a, b, preferred_element_type=F32,
                   precision=lax.Precision.HIGHEST)


def _prep_body(x_ref, aW_ref, ab_ref, o_ref):
    o_ref[...] = _dot(x_ref[...], aW_ref[...]) + ab_ref[...]


def _prep_call(x, aW, ab):
    return pl.pallas_call(
        _prep_body,
        out_shape=jax.ShapeDtypeStruct((N_NODES, C), F32),
    )(x, aW, ab.reshape(1, C))


def _ef_body(d2_ref, W1_ref, b1_ref, W2_ref, b2_ref, o_ref):
    d = jnp.sqrt(d2_ref[...])
    centers = lax.broadcasted_iota(jnp.int32, (1, RBF_BINS), 1).astype(F32) * (
        8.0 / (RBF_BINS - 1))
    gamma = 1.0 / (8.0 / (RBF_BINS - 1))
    rbf = jnp.exp(-gamma * (d - centers) ** 2)
    h = _dot(rbf, W1_ref[...]) + b1_ref[...]
    sp = jnp.maximum(h, 0.0) + jnp.log1p(jnp.exp(-jnp.abs(h)))
    o_ref[...] = _dot(sp, W2_ref[...]) + b2_ref[...]


def _ef_call(d2, W1, b1, W2, b2):
    B = 512
    grid = (N_EDGES // B,)
    return pl.pallas_call(
        _ef_body,
        grid=grid,
        in_specs=[
            pl.BlockSpec((B, 1), lambda i: (i, 0)),
            pl.BlockSpec((RBF_BINS, C), lambda i: (0, 0)),
            pl.BlockSpec((1, C), lambda i: (0, 0)),
            pl.BlockSpec((C, C), lambda i: (0, 0)),
            pl.BlockSpec((1, C), lambda i: (0, 0)),
        ],
        out_specs=pl.BlockSpec((B, C), lambda i: (i, 0)),
        out_shape=jax.ShapeDtypeStruct((N_EDGES, C), F32),
    )(d2, W1, b1.reshape(1, C), W2, b2.reshape(1, C))


def _qkv_body(nf_ref, Wq_ref, bq_ref, Wk_ref, bk_ref, Wv_ref, bv_ref,
              dst_ref, src_ref):
    nf = nf_ref[...]
    q = _dot(nf, Wq_ref[...]) + bq_ref[...]
    k = _dot(nf, Wk_ref[...]) + bk_ref[...]
    v = _dot(nf, Wv_ref[...]) + bv_ref[...]
    dst_ref[:, 0:HC] = q
    dst_ref[:, HC:2 * HC] = q * k
    dst_ref[:, 2 * HC:3 * HC] = v
    src_ref[:, 0:HC] = k
    src_ref[:, HC:2 * HC] = v


def _qkv_call(nf, Wq, bq, Wk, bk, Wv, bv):
    B = 1024
    grid = (N_NODES // B,)
    wspec = pl.BlockSpec((C, HC), lambda i: (0, 0))
    bspec = pl.BlockSpec((1, HC), lambda i: (0, 0))
    return pl.pallas_call(
        _qkv_body,
        grid=grid,
        in_specs=[pl.BlockSpec((B, C), lambda i: (i, 0)),
                  wspec, bspec, wspec, bspec, wspec, bspec],
        out_specs=(pl.BlockSpec((B, 3 * HC), lambda i: (i, 0)),
                   pl.BlockSpec((B, 2 * HC), lambda i: (i, 0))),
        out_shape=(jax.ShapeDtypeStruct((N_NODES, 3 * HC), F32),
                   jax.ShapeDtypeStruct((N_NODES, 2 * HC), F32)),
    )(nf, Wq, bq.reshape(1, HC), Wk, bk.reshape(1, HC), Wv, bv.reshape(1, HC))


def _edge_body(gdst_ref, gsrc_ref, ef_ref,
               We_ref, be_ref, Wm_ref, bm_ref, Wmsg_ref, bmsg_ref,
               lng_ref, lnb_ref, lnmg_ref, lnmb_ref, o_ref):
    scale = 1.0 / np.sqrt(3.0 * C)
    e_full = _dot(ef_ref[...], We_ref[...]) + be_ref[...]
    Wm = Wm_ref[...]
    Wmsg = Wmsg_ref[...]
    bm = bm_ref[...]
    lng = lng_ref[...]
    lnb = lnb_ref[...]
    for h in range(H):
        sl = slice(h * C, (h + 1) * C)
        q_h = gdst_ref[:, h * C:(h + 1) * C]
        g1_h = gdst_ref[:, HC + h * C:HC + (h + 1) * C]
        vd_h = gdst_ref[:, 2 * HC + h * C:2 * HC + (h + 1) * C]
        ks_h = gsrc_ref[:, h * C:(h + 1) * C]
        vs_h = gsrc_ref[:, HC + h * C:HC + (h + 1) * C]
        e_h = e_full[:, sl]
        a1 = g1_h * scale
        a2 = (q_h * ks_h) * scale
        a3 = (q_h * e_h) * scale
        m = (jnp.sum(a1, 1, keepdims=True) + jnp.sum(a2, 1, keepdims=True)
             + jnp.sum(a3, 1, keepdims=True)) * (1.0 / (3 * C))
        d1 = a1 - m
        d2 = a2 - m
        d3 = a3 - m
        var = (jnp.sum(d1 * d1, 1, keepdims=True)
               + jnp.sum(d2 * d2, 1, keepdims=True)
               + jnp.sum(d3 * d3, 1, keepdims=True)) * (1.0 / (3 * C))
        rstd = lax.rsqrt(var + 1e-5)
        s1 = jax.nn.sigmoid(d1 * rstd * lng[:, 0:C] + lnb[:, 0:C])
        s2 = jax.nn.sigmoid(d2 * rstd * lng[:, C:2 * C] + lnb[:, C:2 * C])
        s3 = jax.nn.sigmoid(d3 * rstd * lng[:, 2 * C:3 * C] + lnb[:, 2 * C:3 * C])
        sig = jnp.concatenate([s1, s2, s3], axis=1)
        msg = (_dot(vd_h, Wm[0:C, :]) + _dot(vs_h, Wm[C:2 * C, :])
               + _dot(e_h, Wm[2 * C:3 * C, :]) + bm)
        msg = msg * sig
        msg2 = _dot(msg, Wmsg) + bmsg_ref[...]
        m2 = jnp.mean(msg2, 1, keepdims=True)
        dv = msg2 - m2
        v2 = jnp.mean(dv * dv, 1, keepdims=True)
        res = dv * lax.rsqrt(v2 + 1e-5) * lnmg_ref[...] + lnmb_ref[...]
        o_ref[sl, :] = res.T


def _edge_call(gdst, gsrc, ef, We, be, Wm, bm, Wmsg, bmsg, lng, lnb, lnmg, lnmb):
    B = 512
    grid = (N_EDGES // B,)
    return pl.pallas_call(
        _edge_body,
        grid=grid,
        in_specs=[
            pl.BlockSpec((B, 3 * HC), lambda i: (i, 0)),
            pl.BlockSpec((B, 2 * HC), lambda i: (i, 0)),
            pl.BlockSpec((B, C), lambda i: (i, 0)),
            pl.BlockSpec((C, HC), lambda i: (0, 0)),
            pl.BlockSpec((1, HC), lambda i: (0, 0)),
            pl.BlockSpec((3 * C, 3 * C), lambda i: (0, 0)),
            pl.BlockSpec((1, 3 * C), lambda i: (0, 0)),
            pl.BlockSpec((3 * C, C), lambda i: (0, 0)),
            pl.BlockSpec((1, C), lambda i: (0, 0)),
            pl.BlockSpec((1, 3 * C), lambda i: (0, 0)),
            pl.BlockSpec((1, 3 * C), lambda i: (0, 0)),
            pl.BlockSpec((1, C), lambda i: (0, 0)),
            pl.BlockSpec((1, C), lambda i: (0, 0)),
        ],
        out_specs=pl.BlockSpec((HC, B), lambda i: (0, i)),
        out_shape=jax.ShapeDtypeStruct((HC, N_EDGES), F32),
    )(gdst, gsrc, ef, We, be.reshape(1, HC), Wm, bm.reshape(1, 3 * C),
      Wmsg, bmsg.reshape(1, C), lng.reshape(1, 3 * C), lnb.reshape(1, 3 * C),
      lnmg.reshape(1, C), lnmb.reshape(1, C))


def _epi_body(agg_ref, nf_ref, Wc_ref, bc_ref, bng_ref, bnb_ref,
              Ws_ref, bs_ref, o_ref):
    out = lax.dot_general(agg_ref[...], Wc_ref[...],
                          (((0,), (0,)), ((), ())),
                          preferred_element_type=F32,
                          precision=lax.Precision.HIGHEST) + bc_ref[...]
    mu = jnp.mean(out, 0, keepdims=True)
    dv = out - mu
    var = jnp.mean(dv * dv, 0, keepdims=True)
    out = dv * lax.rsqrt(var + 1e-5) * bng_ref[...] + bnb_ref[...]
    out = out * jax.nn.sigmoid(out)
    o_ref[...] = out + _dot(nf_ref[...], Ws_ref[...]) + bs_ref[...]


def _epi_call(agg, nf, Wc, bc, bng, bnb, Ws, bs):
    return pl.pallas_call(
        _epi_body,
        out_shape=jax.ShapeDtypeStruct((N_NODES, C), F32),
    )(agg, nf, Wc, bc.reshape(1, C), bng.reshape(1, C), bnb.reshape(1, C),
      Ws, bs.reshape(1, C))


def _pool_body(nf_ref, batch_ref, fcW_ref, fcb_ref, oW_ref, ob_ref, o_ref):
    ids = lax.broadcasted_iota(jnp.int32, (N_GRAPHS, 1), 0)
    oh = (batch_ref[...] == ids).astype(F32)
    counts = jnp.sum(oh, 1, keepdims=True)
    pooled = _dot(oh, nf_ref[...]) / jnp.maximum(counts, 1.0)
    feat = _dot(pooled, fcW_ref[...]) + fcb_ref[...]
    feat = feat * jax.nn.sigmoid(feat)
    o_ref[...] = _dot(feat, oW_ref[...]) + ob_ref[...]


def _pool_call(nf, batch_row, fcW, fcb, oW, ob):
    return pl.pallas_call(
        _pool_body,
        out_shape=jax.ShapeDtypeStruct((N_GRAPHS, 1), F32),
    )(nf, batch_row, fcW, fcb.reshape(1, C), oW, ob.reshape(1, 1))


def _sc_gather(table, idx, width, gch):
    epw = N_EDGES // SC_WORKERS

    @functools.partial(
        pl.kernel,
        mesh=plsc.VectorSubcoreMesh(**_SC_MESH),
        out_type=jax.ShapeDtypeStruct((N_EDGES, width), F32),
        scratch_types=[
            pltpu.VMEM((gch,), jnp.int32),
            pltpu.VMEM((gch, width), F32),
            pltpu.SemaphoreType.DMA,
        ],
    )
    def k(table_hbm, idx_hbm, out_hbm, idx_v, rows_v, sem):
        wid = lax.axis_index("s") * SC_CORES + lax.axis_index("c")
        base = wid * epw

        def body(i, carry):
            off = base + i * gch
            pltpu.sync_copy(idx_hbm.at[pl.ds(off, gch)], idx_v)
            pltpu.async_copy(table_hbm.at[idx_v], rows_v, sem).wait()
            pltpu.sync_copy(rows_v, out_hbm.at[pl.ds(off, gch)])
            return carry

        lax.fori_loop(0, epw // gch, body, 0)

    return k(table, idx)


def _sc_scatter_add(msg2_t, idx, zeros):
    rows = 16
    sch = 512
    nch = N_EDGES // sch

    @functools.partial(
        pl.kernel,
        mesh=plsc.VectorSubcoreMesh(**_SC_MESH),
        out_type=jax.ShapeDtypeStruct((HC, N_NODES), F32),
        scratch_types=[
            pltpu.VMEM((sch,), jnp.int32),
            pltpu.VMEM((sch,), jnp.int32),
            pltpu.VMEM((rows, sch), F32),
            pltpu.VMEM((rows, sch), F32),
            pltpu.VMEM((rows, N_NODES), F32),
            pltpu.SemaphoreType.DMA,
            pltpu.SemaphoreType.DMA,
            pltpu.SemaphoreType.DMA,
            pltpu.SemaphoreType.DMA,
        ],
        compiler_params=pltpu.CompilerParams(needs_layout_passes=False),
    )
    def k(msg_hbm, idx_hbm, zeros_hbm, agg_hbm,
          idx0, idx1, rv0, rv1, acc_v, is0, is1, ds0, ds1):
        wid = lax.axis_index("s") * SC_CORES + lax.axis_index("c")
        r0 = wid * rows
        pltpu.sync_copy(zeros_hbm, acc_v)
        lanes = lax.iota(jnp.int32, 16)
        idx_bufs = (idx0, idx1)
        row_bufs = (rv0, rv1)
        isems = (is0, is1)
        dsems = (ds0, ds1)

        def start(i, b):
            off = i * sch
            pltpu.async_copy(idx_hbm.at[pl.ds(off, sch)], idx_bufs[b],
                             isems[b])
            pltpu.async_copy(msg_hbm.at[pl.ds(r0, rows), pl.ds(off, sch)],
                             row_bufs[b], dsems[b])

        def wait(b):
            pltpu.make_async_copy(idx_hbm.at[pl.ds(0, sch)], idx_bufs[b],
                                  isems[b]).wait()
            pltpu.make_async_copy(msg_hbm.at[pl.ds(r0, rows), pl.ds(0, sch)],
                                  row_bufs[b], dsems[b]).wait()

        def compute(b):
            idx_v = idx_bufs[b]
            rows_v = row_bufs[b]

            def group(g, carry2):
                dstv = idx_v[pl.ds(g * 16, 16)]
                for j in range(16):
                    jv = jnp.full((16,), j, jnp.int32)
                    vals = plsc.load_gather(
                        rows_v, [lanes, jnp.full((16,), g * 16 + j, jnp.int32)])
                    dstb = jnp.take(dstv, jv, mode="wrap")
                    plsc.addupdate_scatter(acc_v, [lanes, dstb], vals)
                return carry2

            lax.fori_loop(0, sch // 16, group, 0)

        start(0, 0)

        def body2(i2, carry):
            i = i2 * 2

            @pl.when(i + 1 < nch)
            def _():
                start(i + 1, 1)

            wait(0)
            compute(0)

            @pl.when(i + 2 < nch)
            def _():
                start(i + 2, 0)

            @pl.when(i + 1 < nch)
            def _():
                wait(1)
                compute(1)

            return carry

        lax.fori_loop(0, (nch + 1) // 2, body2, 0)
        pltpu.sync_copy(acc_v, agg_hbm.at[pl.ds(r0, rows)])

    return k(msg2_t, idx, zeros)


def kernel(x, edge_index, edge_attr, batch, params):
    p = params
    src = edge_index[0].astype(jnp.int32)
    dst = edge_index[1].astype(jnp.int32)
    batch_row = batch.astype(jnp.int32).reshape(1, N_NODES)
    d2 = jnp.sum(edge_attr * edge_attr, axis=1, keepdims=True)
    zeros = jnp.zeros((16, N_NODES), F32)

    nf = _prep_call(x, p['atom_W'], p['atom_b'])
    ef = _ef_call(d2, p['rbf_W1'], p['rbf_b1'], p['rbf_W2'], p['rbf_b2'])

    for l in range(CONV_LAYERS):
        tdst, tsrc = _qkv_call(nf, p['Wq'][l], p['bq'][l], p['Wk'][l],
                               p['bk'][l], p['Wv'][l], p['bv'][l])
        gdst = _sc_gather(tdst, dst, 3 * HC, 64)
        gsrc = _sc_gather(tsrc, src, 2 * HC, 96)
        msg2 = _edge_call(gdst, gsrc, ef, p['We'][l], p['be'][l], p['Wm'][l],
                          p['bm'][l], p['Wmsg'][l], p['bmsg'][l],
                          p['ln_g'][l], p['ln_b'][l], p['lnm_g'][l],
                          p['lnm_b'][l])
        agg = _sc_scatter_add(msg2, dst, zeros)
        nf = _epi_call(agg, nf, p['Wc'][l], p['bc'][l], p['bn_g'][l],
                       p['bn_b'][l], p['Ws'][l], p['bs'][l])

    out = _pool_call(nf, batch_row, p['fc_W'], p['fc_b'], p['out_W'], p['out_b'])
    return out.reshape(N_GRAPHS)


# speedup vs baseline: 4.5722x; 1.4209x over previous
import functools

import jax
import jax.numpy as jnp
import numpy as np
from jax import lax
from jax.experimental import pallas as pl
from jax.experimental.pallas import tpu as pltpu
from jax.experimental.pallas import tpu_sc as plsc

N_NODES = 4096
N_EDGES = 49152
N_GRAPHS = 128
CONV_LAYERS = 5
ATOM_F = 92
RBF_BINS = 128
C = 128
H = 4
HC = H * C

F32 = jnp.float32

SC_CORES = 2
SC_SUBCORES = 16
SC_WORKERS = SC_CORES * SC_SUBCORES

_SC_MESH = dict(core_axis_name="c", subcore_axis_name="s")


def _dot(a, b):
    return jnp.dot(a, b, preferred_element_type=F32,
                   precision=lax.Precision.HIGHEST)


def _prep_body(x_ref, aW_ref, ab_ref, o_ref):
    o_ref[...] = _dot(x_ref[...], aW_ref[...]) + ab_ref[...]


def _prep_call(x, aW, ab):
    return pl.pallas_call(
        _prep_body,
        out_shape=jax.ShapeDtypeStruct((N_NODES, C), F32),
    )(x, aW, ab.reshape(1, C))


def _ef_body(d2_ref, W1_ref, b1_ref, W2_ref, b2_ref, o_ref):
    d = jnp.sqrt(d2_ref[...])
    centers = lax.broadcasted_iota(jnp.int32, (1, RBF_BINS), 1).astype(F32) * (
        8.0 / (RBF_BINS - 1))
    gamma = 1.0 / (8.0 / (RBF_BINS - 1))
    rbf = jnp.exp(-gamma * (d - centers) ** 2)
    h = _dot(rbf, W1_ref[...]) + b1_ref[...]
    sp = jnp.maximum(h, 0.0) + jnp.log1p(jnp.exp(-jnp.abs(h)))
    o_ref[...] = _dot(sp, W2_ref[...]) + b2_ref[...]


def _ef_call(d2, W1, b1, W2, b2):
    B = 512
    grid = (N_EDGES // B,)
    return pl.pallas_call(
        _ef_body,
        grid=grid,
        in_specs=[
            pl.BlockSpec((B, 1), lambda i: (i, 0)),
            pl.BlockSpec((RBF_BINS, C), lambda i: (0, 0)),
            pl.BlockSpec((1, C), lambda i: (0, 0)),
            pl.BlockSpec((C, C), lambda i: (0, 0)),
            pl.BlockSpec((1, C), lambda i: (0, 0)),
        ],
        out_specs=pl.BlockSpec((B, C), lambda i: (i, 0)),
        out_shape=jax.ShapeDtypeStruct((N_EDGES, C), F32),
    )(d2, W1, b1.reshape(1, C), W2, b2.reshape(1, C))


def _qkv_body(nf_ref, Wq_ref, bq_ref, Wk_ref, bk_ref, Wv_ref, bv_ref,
              dst_ref, src_ref):
    nf = nf_ref[...]
    q = _dot(nf, Wq_ref[...]) + bq_ref[...]
    k = _dot(nf, Wk_ref[...]) + bk_ref[...]
    v = _dot(nf, Wv_ref[...]) + bv_ref[...]
    dst_ref[:, 0:HC] = q
    dst_ref[:, HC:2 * HC] = q * k
    dst_ref[:, 2 * HC:3 * HC] = v
    src_ref[:, 0:HC] = k
    src_ref[:, HC:2 * HC] = v


def _qkv_call(nf, Wq, bq, Wk, bk, Wv, bv):
    B = 1024
    grid = (N_NODES // B,)
    wspec = pl.BlockSpec((C, HC), lambda i: (0, 0))
    bspec = pl.BlockSpec((1, HC), lambda i: (0, 0))
    return pl.pallas_call(
        _qkv_body,
        grid=grid,
        in_specs=[pl.BlockSpec((B, C), lambda i: (i, 0)),
                  wspec, bspec, wspec, bspec, wspec, bspec],
        out_specs=(pl.BlockSpec((B, 3 * HC), lambda i: (i, 0)),
                   pl.BlockSpec((B, 2 * HC), lambda i: (i, 0))),
        out_shape=(jax.ShapeDtypeStruct((N_NODES, 3 * HC), F32),
                   jax.ShapeDtypeStruct((N_NODES, 2 * HC), F32)),
    )(nf, Wq, bq.reshape(1, HC), Wk, bk.reshape(1, HC), Wv, bv.reshape(1, HC))


def _edge_body(gdst_ref, gsrc_ref, ef_ref,
               We_ref, be_ref, Wm_ref, bm_ref, Wmsg_ref, bmsg_ref,
               lng_ref, lnb_ref, lnmg_ref, lnmb_ref, o_ref):
    scale = 1.0 / np.sqrt(3.0 * C)
    e_full = _dot(ef_ref[...], We_ref[...]) + be_ref[...]
    Wm = Wm_ref[...]
    Wmsg = Wmsg_ref[...]
    bm = bm_ref[...]
    lng = lng_ref[...]
    lnb = lnb_ref[...]
    for h in range(H):
        sl = slice(h * C, (h + 1) * C)
        q_h = gdst_ref[:, h * C:(h + 1) * C]
        g1_h = gdst_ref[:, HC + h * C:HC + (h + 1) * C]
        vd_h = gdst_ref[:, 2 * HC + h * C:2 * HC + (h + 1) * C]
        ks_h = gsrc_ref[:, h * C:(h + 1) * C]
        vs_h = gsrc_ref[:, HC + h * C:HC + (h + 1) * C]
        e_h = e_full[:, sl]
        a1 = g1_h * scale
        a2 = (q_h * ks_h) * scale
        a3 = (q_h * e_h) * scale
        m = (jnp.sum(a1, 1, keepdims=True) + jnp.sum(a2, 1, keepdims=True)
             + jnp.sum(a3, 1, keepdims=True)) * (1.0 / (3 * C))
        d1 = a1 - m
        d2 = a2 - m
        d3 = a3 - m
        var = (jnp.sum(d1 * d1, 1, keepdims=True)
               + jnp.sum(d2 * d2, 1, keepdims=True)
               + jnp.sum(d3 * d3, 1, keepdims=True)) * (1.0 / (3 * C))
        rstd = lax.rsqrt(var + 1e-5)
        s1 = jax.nn.sigmoid(d1 * rstd * lng[:, 0:C] + lnb[:, 0:C])
        s2 = jax.nn.sigmoid(d2 * rstd * lng[:, C:2 * C] + lnb[:, C:2 * C])
        s3 = jax.nn.sigmoid(d3 * rstd * lng[:, 2 * C:3 * C] + lnb[:, 2 * C:3 * C])
        sig = jnp.concatenate([s1, s2, s3], axis=1)
        msg = (_dot(vd_h, Wm[0:C, :]) + _dot(vs_h, Wm[C:2 * C, :])
               + _dot(e_h, Wm[2 * C:3 * C, :]) + bm)
        msg = msg * sig
        msg2 = _dot(msg, Wmsg) + bmsg_ref[...]
        m2 = jnp.mean(msg2, 1, keepdims=True)
        dv = msg2 - m2
        v2 = jnp.mean(dv * dv, 1, keepdims=True)
        res = dv * lax.rsqrt(v2 + 1e-5) * lnmg_ref[...] + lnmb_ref[...]
        o_ref[sl, :] = res.T


def _edge_call(gdst, gsrc, ef, We, be, Wm, bm, Wmsg, bmsg, lng, lnb, lnmg, lnmb):
    B = 512
    grid = (N_EDGES // B,)
    return pl.pallas_call(
        _edge_body,
        grid=grid,
        in_specs=[
            pl.BlockSpec((B, 3 * HC), lambda i: (i, 0)),
            pl.BlockSpec((B, 2 * HC), lambda i: (i, 0)),
            pl.BlockSpec((B, C), lambda i: (i, 0)),
            pl.BlockSpec((C, HC), lambda i: (0, 0)),
            pl.BlockSpec((1, HC), lambda i: (0, 0)),
            pl.BlockSpec((3 * C, 3 * C), lambda i: (0, 0)),
            pl.BlockSpec((1, 3 * C), lambda i: (0, 0)),
            pl.BlockSpec((3 * C, C), lambda i: (0, 0)),
            pl.BlockSpec((1, C), lambda i: (0, 0)),
            pl.BlockSpec((1, 3 * C), lambda i: (0, 0)),
            pl.BlockSpec((1, 3 * C), lambda i: (0, 0)),
            pl.BlockSpec((1, C), lambda i: (0, 0)),
            pl.BlockSpec((1, C), lambda i: (0, 0)),
        ],
        out_specs=pl.BlockSpec((HC, B), lambda i: (0, i)),
        out_shape=jax.ShapeDtypeStruct((HC, N_EDGES), F32),
    )(gdst, gsrc, ef, We, be.reshape(1, HC), Wm, bm.reshape(1, 3 * C),
      Wmsg, bmsg.reshape(1, C), lng.reshape(1, 3 * C), lnb.reshape(1, 3 * C),
      lnmg.reshape(1, C), lnmb.reshape(1, C))


def _epi_body(agg_ref, nf_ref, Wc_ref, bc_ref, bng_ref, bnb_ref,
              Ws_ref, bs_ref, o_ref):
    out = lax.dot_general(agg_ref[...], Wc_ref[...],
                          (((0,), (0,)), ((), ())),
                          preferred_element_type=F32,
                          precision=lax.Precision.HIGHEST) + bc_ref[...]
    mu = jnp.mean(out, 0, keepdims=True)
    dv = out - mu
    var = jnp.mean(dv * dv, 0, keepdims=True)
    out = dv * lax.rsqrt(var + 1e-5) * bng_ref[...] + bnb_ref[...]
    out = out * jax.nn.sigmoid(out)
    o_ref[...] = out + _dot(nf_ref[...], Ws_ref[...]) + bs_ref[...]


def _epi_call(agg, nf, Wc, bc, bng, bnb, Ws, bs):
    return pl.pallas_call(
        _epi_body,
        out_shape=jax.ShapeDtypeStruct((N_NODES, C), F32),
    )(agg, nf, Wc, bc.reshape(1, C), bng.reshape(1, C), bnb.reshape(1, C),
      Ws, bs.reshape(1, C))


def _pool_body(nf_ref, batch_ref, fcW_ref, fcb_ref, oW_ref, ob_ref, o_ref):
    ids = lax.broadcasted_iota(jnp.int32, (N_GRAPHS, 1), 0)
    oh = (batch_ref[...] == ids).astype(F32)
    counts = jnp.sum(oh, 1, keepdims=True)
    pooled = _dot(oh, nf_ref[...]) / jnp.maximum(counts, 1.0)
    feat = _dot(pooled, fcW_ref[...]) + fcb_ref[...]
    feat = feat * jax.nn.sigmoid(feat)
    o_ref[...] = _dot(feat, oW_ref[...]) + ob_ref[...]


def _pool_call(nf, batch_row, fcW, fcb, oW, ob):
    return pl.pallas_call(
        _pool_body,
        out_shape=jax.ShapeDtypeStruct((N_GRAPHS, 1), F32),
    )(nf, batch_row, fcW, fcb.reshape(1, C), oW, ob.reshape(1, 1))


def _sc_gather(table, idx, width, gch):
    epw = N_EDGES // SC_WORKERS

    @functools.partial(
        pl.kernel,
        mesh=plsc.VectorSubcoreMesh(**_SC_MESH),
        out_type=jax.ShapeDtypeStruct((N_EDGES, width), F32),
        scratch_types=[
            pltpu.VMEM((gch,), jnp.int32),
            pltpu.VMEM((gch, width), F32),
            pltpu.SemaphoreType.DMA,
        ],
    )
    def k(table_hbm, idx_hbm, out_hbm, idx_v, rows_v, sem):
        wid = lax.axis_index("s") * SC_CORES + lax.axis_index("c")
        base = wid * epw

        def body(i, carry):
            off = base + i * gch
            pltpu.sync_copy(idx_hbm.at[pl.ds(off, gch)], idx_v)
            pltpu.async_copy(table_hbm.at[idx_v], rows_v, sem).wait()
            pltpu.sync_copy(rows_v, out_hbm.at[pl.ds(off, gch)])
            return carry

        lax.fori_loop(0, epw // gch, body, 0)

    return k(table, idx)


def _sc_scatter_add(msg2_t, idx, zeros):
    rows = 16
    sch = 512
    nch = N_EDGES // sch

    @functools.partial(
        pl.kernel,
        mesh=plsc.VectorSubcoreMesh(**_SC_MESH),
        out_type=jax.ShapeDtypeStruct((HC, N_NODES), F32),
        scratch_types=[
            pltpu.VMEM((sch,), jnp.int32),
            pltpu.VMEM((sch,), jnp.int32),
            pltpu.VMEM((rows, sch), F32),
            pltpu.VMEM((rows, sch), F32),
            pltpu.VMEM((rows, N_NODES), F32),
            pltpu.SemaphoreType.DMA,
            pltpu.SemaphoreType.DMA,
            pltpu.SemaphoreType.DMA,
            pltpu.SemaphoreType.DMA,
        ],
        compiler_params=pltpu.CompilerParams(needs_layout_passes=False),
    )
    def k(msg_hbm, idx_hbm, zeros_hbm, agg_hbm,
          idx0, idx1, rv0, rv1, acc_v, is0, is1, ds0, ds1):
        wid = lax.axis_index("s") * SC_CORES + lax.axis_index("c")
        r0 = wid * rows
        pltpu.sync_copy(zeros_hbm, acc_v)
        lanes = lax.iota(jnp.int32, 16)
        idx_bufs = (idx0, idx1)
        row_bufs = (rv0, rv1)
        isems = (is0, is1)
        dsems = (ds0, ds1)

        def start(i, b):
            off = i * sch
            pltpu.async_copy(idx_hbm.at[pl.ds(off, sch)], idx_bufs[b],
                             isems[b])
            pltpu.async_copy(msg_hbm.at[pl.ds(r0, rows), pl.ds(off, sch)],
                             row_bufs[b], dsems[b])

        def wait(b):
            pltpu.make_async_copy(idx_hbm.at[pl.ds(0, sch)], idx_bufs[b],
                                  isems[b]).wait()
            pltpu.make_async_copy(msg_hbm.at[pl.ds(r0, rows), pl.ds(0, sch)],
                                  row_bufs[b], dsems[b]).wait()

        def compute(b):
            idx_v = idx_bufs[b]
            rows_v = row_bufs[b]

            def group(g, carry2):
                dstv = idx_v[pl.ds(g * 16, 16)]
                for f in range(rows):
                    vals = rows_v[f, pl.ds(g * 16, 16)]
                    plsc.addupdate_scatter(
                        acc_v, [jnp.full((16,), f, jnp.int32), dstv], vals)
                return carry2

            lax.fori_loop(0, sch // 16, group, 0)

        start(0, 0)

        def body2(i2, carry):
            i = i2 * 2

            @pl.when(i + 1 < nch)
            def _():
                start(i + 1, 1)

            wait(0)
            compute(0)

            @pl.when(i + 2 < nch)
            def _():
                start(i + 2, 0)

            @pl.when(i + 1 < nch)
            def _():
                wait(1)
                compute(1)

            return carry

        lax.fori_loop(0, (nch + 1) // 2, body2, 0)
        pltpu.sync_copy(acc_v, agg_hbm.at[pl.ds(r0, rows)])

    return k(msg2_t, idx, zeros)


def kernel(x, edge_index, edge_attr, batch, params):
    p = params
    src = edge_index[0].astype(jnp.int32)
    dst = edge_index[1].astype(jnp.int32)
    batch_row = batch.astype(jnp.int32).reshape(1, N_NODES)
    d2 = jnp.sum(edge_attr * edge_attr, axis=1, keepdims=True)
    zeros = jnp.zeros((16, N_NODES), F32)

    nf = _prep_call(x, p['atom_W'], p['atom_b'])
    ef = _ef_call(d2, p['rbf_W1'], p['rbf_b1'], p['rbf_W2'], p['rbf_b2'])

    for l in range(CONV_LAYERS):
        tdst, tsrc = _qkv_call(nf, p['Wq'][l], p['bq'][l], p['Wk'][l],
                               p['bk'][l], p['Wv'][l], p['bv'][l])
        gdst = _sc_gather(tdst, dst, 3 * HC, 64)
        gsrc = _sc_gather(tsrc, src, 2 * HC, 96)
        msg2 = _edge_call(gdst, gsrc, ef, p['We'][l], p['be'][l], p['Wm'][l],
                          p['bm'][l], p['Wmsg'][l], p['bmsg'][l],
                          p['ln_g'][l], p['ln_b'][l], p['lnm_g'][l],
                          p['lnm_b'][l])
        agg = _sc_scatter_add(msg2, dst, zeros)
        nf = _epi_call(agg, nf, p['Wc'][l], p['bc'][l], p['bn_g'][l],
                       p['bn_b'][l], p['Ws'][l], p['bs'][l])

    out = _pool_call(nf, batch_row, p['fc_W'], p['fc_b'], p['out_W'], p['out_b'])
    return out.reshape(N_GRAPHS)


# speedup vs baseline: 6.6510x; 1.4547x over previous
import functools

import jax
import jax.numpy as jnp
import numpy as np
from jax import lax
from jax.experimental import pallas as pl
from jax.experimental.pallas import tpu as pltpu
from jax.experimental.pallas import tpu_sc as plsc

N_NODES = 4096
N_EDGES = 49152
N_GRAPHS = 128
CONV_LAYERS = 5
ATOM_F = 92
RBF_BINS = 128
C = 128
H = 4
HC = H * C

F32 = jnp.float32

SC_CORES = 2
SC_SUBCORES = 16
SC_WORKERS = SC_CORES * SC_SUBCORES

_SC_MESH = dict(core_axis_name="c", subcore_axis_name="s")


def _dot(a, b, precision=lax.Precision.HIGHEST):
    return jnp.dot(a, b, preferred_element_type=F32, precision=precision)


def _dot_h(a, b):
    ah = a.astype(jnp.bfloat16)
    al = (a - ah.astype(F32)).astype(jnp.bfloat16)
    bh = b.astype(jnp.bfloat16)
    bl = (b - bh.astype(F32)).astype(jnp.bfloat16)
    d = functools.partial(jnp.dot, preferred_element_type=F32)
    return d(ah, bh) + (d(ah, bl) + d(al, bh))


def _prep_body(x_ref, aW_ref, ab_ref, o_ref):
    o_ref[...] = _dot(x_ref[...], aW_ref[...]) + ab_ref[...]


def _prep_call(x, aW, ab):
    return pl.pallas_call(
        _prep_body,
        out_shape=jax.ShapeDtypeStruct((N_NODES, C), F32),
    )(x, aW, ab.reshape(1, C))


def _ef_body(d2_ref, W1_ref, b1_ref, W2_ref, b2_ref, o_ref):
    d = jnp.sqrt(d2_ref[...])
    centers = lax.broadcasted_iota(jnp.int32, (1, RBF_BINS), 1).astype(F32) * (
        8.0 / (RBF_BINS - 1))
    gamma = 1.0 / (8.0 / (RBF_BINS - 1))
    rbf = jnp.exp(-gamma * (d - centers) ** 2)
    h = _dot(rbf, W1_ref[...]) + b1_ref[...]
    sp = jnp.maximum(h, 0.0) + jnp.log1p(jnp.exp(-jnp.abs(h)))
    o_ref[...] = _dot(sp, W2_ref[...]) + b2_ref[...]


def _ef_call(d2, W1, b1, W2, b2):
    B = 512
    grid = (N_EDGES // B,)
    return pl.pallas_call(
        _ef_body,
        grid=grid,
        in_specs=[
            pl.BlockSpec((B, 1), lambda i: (i, 0)),
            pl.BlockSpec((RBF_BINS, C), lambda i: (0, 0)),
            pl.BlockSpec((1, C), lambda i: (0, 0)),
            pl.BlockSpec((C, C), lambda i: (0, 0)),
            pl.BlockSpec((1, C), lambda i: (0, 0)),
        ],
        out_specs=pl.BlockSpec((B, C), lambda i: (i, 0)),
        out_shape=jax.ShapeDtypeStruct((N_EDGES, C), F32),
    )(d2, W1, b1.reshape(1, C), W2, b2.reshape(1, C))


def _qkv_body(nf_ref, Wq_ref, bq_ref, Wk_ref, bk_ref, Wv_ref, bv_ref,
              dst_ref, src_ref):
    nf = nf_ref[...]
    q = _dot(nf, Wq_ref[...]) + bq_ref[...]
    k = _dot(nf, Wk_ref[...]) + bk_ref[...]
    v = _dot(nf, Wv_ref[...]) + bv_ref[...]
    dst_ref[:, 0:HC] = q
    dst_ref[:, HC:2 * HC] = q * k
    dst_ref[:, 2 * HC:3 * HC] = v
    src_ref[:, 0:HC] = k
    src_ref[:, HC:2 * HC] = v


def _qkv_call(nf, Wq, bq, Wk, bk, Wv, bv):
    B = 1024
    grid = (N_NODES // B,)
    wspec = pl.BlockSpec((C, HC), lambda i: (0, 0))
    bspec = pl.BlockSpec((1, HC), lambda i: (0, 0))
    return pl.pallas_call(
        _qkv_body,
        grid=grid,
        in_specs=[pl.BlockSpec((B, C), lambda i: (i, 0)),
                  wspec, bspec, wspec, bspec, wspec, bspec],
        out_specs=(pl.BlockSpec((B, 3 * HC), lambda i: (i, 0)),
                   pl.BlockSpec((B, 2 * HC), lambda i: (i, 0))),
        out_shape=(jax.ShapeDtypeStruct((N_NODES, 3 * HC), F32),
                   jax.ShapeDtypeStruct((N_NODES, 2 * HC), F32)),
    )(nf, Wq, bq.reshape(1, HC), Wk, bk.reshape(1, HC), Wv, bv.reshape(1, HC))


def _edge_body(gdst_ref, gsrc_ref, ef_ref,
               We_ref, be_ref, Wm_ref, bm_ref, Wmsg_ref, bmsg_ref,
               lng_ref, lnb_ref, lnmg_ref, lnmb_ref, o_ref):
    scale = 1.0 / np.sqrt(3.0 * C)
    e_full = _dot_h(ef_ref[...], We_ref[...]) + be_ref[...]
    Wm = Wm_ref[...]
    Wmsg = Wmsg_ref[...]
    bm = bm_ref[...]
    lng = lng_ref[...]
    lnb = lnb_ref[...]
    for h in range(H):
        sl = slice(h * C, (h + 1) * C)
        q_h = gdst_ref[:, h * C:(h + 1) * C]
        g1_h = gdst_ref[:, HC + h * C:HC + (h + 1) * C]
        vd_h = gdst_ref[:, 2 * HC + h * C:2 * HC + (h + 1) * C]
        ks_h = gsrc_ref[:, h * C:(h + 1) * C]
        vs_h = gsrc_ref[:, HC + h * C:HC + (h + 1) * C]
        e_h = e_full[:, sl]
        a1 = g1_h * scale
        a2 = (q_h * ks_h) * scale
        a3 = (q_h * e_h) * scale
        m = (jnp.sum(a1, 1, keepdims=True) + jnp.sum(a2, 1, keepdims=True)
             + jnp.sum(a3, 1, keepdims=True)) * (1.0 / (3 * C))
        d1 = a1 - m
        d2 = a2 - m
        d3 = a3 - m
        var = (jnp.sum(d1 * d1, 1, keepdims=True)
               + jnp.sum(d2 * d2, 1, keepdims=True)
               + jnp.sum(d3 * d3, 1, keepdims=True)) * (1.0 / (3 * C))
        rstd = lax.rsqrt(var + 1e-5)
        s1 = jax.nn.sigmoid(d1 * rstd * lng[:, 0:C] + lnb[:, 0:C])
        s2 = jax.nn.sigmoid(d2 * rstd * lng[:, C:2 * C] + lnb[:, C:2 * C])
        s3 = jax.nn.sigmoid(d3 * rstd * lng[:, 2 * C:3 * C] + lnb[:, 2 * C:3 * C])
        sig = jnp.concatenate([s1, s2, s3], axis=1)
        msg = (_dot_h(vd_h, Wm[0:C, :]) + _dot_h(vs_h, Wm[C:2 * C, :])
               + _dot_h(e_h, Wm[2 * C:3 * C, :]) + bm)
        msg = msg * sig
        msg2 = _dot_h(msg, Wmsg) + bmsg_ref[...]
        m2 = jnp.mean(msg2, 1, keepdims=True)
        dv = msg2 - m2
        v2 = jnp.mean(dv * dv, 1, keepdims=True)
        res = dv * lax.rsqrt(v2 + 1e-5) * lnmg_ref[...] + lnmb_ref[...]
        o_ref[sl, :] = res.T


def _edge_call(gdst, gsrc, ef, We, be, Wm, bm, Wmsg, bmsg, lng, lnb, lnmg, lnmb):
    B = 512
    grid = (N_EDGES // B,)
    return pl.pallas_call(
        _edge_body,
        grid=grid,
        in_specs=[
            pl.BlockSpec((B, 3 * HC), lambda i: (i, 0)),
            pl.BlockSpec((B, 2 * HC), lambda i: (i, 0)),
            pl.BlockSpec((B, C), lambda i: (i, 0)),
            pl.BlockSpec((C, HC), lambda i: (0, 0)),
            pl.BlockSpec((1, HC), lambda i: (0, 0)),
            pl.BlockSpec((3 * C, 3 * C), lambda i: (0, 0)),
            pl.BlockSpec((1, 3 * C), lambda i: (0, 0)),
            pl.BlockSpec((3 * C, C), lambda i: (0, 0)),
            pl.BlockSpec((1, C), lambda i: (0, 0)),
            pl.BlockSpec((1, 3 * C), lambda i: (0, 0)),
            pl.BlockSpec((1, 3 * C), lambda i: (0, 0)),
            pl.BlockSpec((1, C), lambda i: (0, 0)),
            pl.BlockSpec((1, C), lambda i: (0, 0)),
        ],
        out_specs=pl.BlockSpec((HC, B), lambda i: (0, i)),
        out_shape=jax.ShapeDtypeStruct((HC, N_EDGES), F32),
    )(gdst, gsrc, ef, We, be.reshape(1, HC), Wm, bm.reshape(1, 3 * C),
      Wmsg, bmsg.reshape(1, C), lng.reshape(1, 3 * C), lnb.reshape(1, 3 * C),
      lnmg.reshape(1, C), lnmb.reshape(1, C))


def _epi_body(agg_ref, nf_ref, Wc_ref, bc_ref, bng_ref, bnb_ref,
              Ws_ref, bs_ref, o_ref):
    out = lax.dot_general(agg_ref[...], Wc_ref[...],
                          (((0,), (0,)), ((), ())),
                          preferred_element_type=F32,
                          precision=lax.Precision.HIGHEST) + bc_ref[...]
    mu = jnp.mean(out, 0, keepdims=True)
    dv = out - mu
    var = jnp.mean(dv * dv, 0, keepdims=True)
    out = dv * lax.rsqrt(var + 1e-5) * bng_ref[...] + bnb_ref[...]
    out = out * jax.nn.sigmoid(out)
    o_ref[...] = out + _dot(nf_ref[...], Ws_ref[...]) + bs_ref[...]


def _epi_call(agg, nf, Wc, bc, bng, bnb, Ws, bs):
    return pl.pallas_call(
        _epi_body,
        out_shape=jax.ShapeDtypeStruct((N_NODES, C), F32),
    )(agg, nf, Wc, bc.reshape(1, C), bng.reshape(1, C), bnb.reshape(1, C),
      Ws, bs.reshape(1, C))


def _pool_body(nf_ref, batch_ref, fcW_ref, fcb_ref, oW_ref, ob_ref, o_ref):
    ids = lax.broadcasted_iota(jnp.int32, (N_GRAPHS, 1), 0)
    oh = (batch_ref[...] == ids).astype(F32)
    counts = jnp.sum(oh, 1, keepdims=True)
    pooled = _dot(oh, nf_ref[...]) / jnp.maximum(counts, 1.0)
    feat = _dot(pooled, fcW_ref[...]) + fcb_ref[...]
    feat = feat * jax.nn.sigmoid(feat)
    o_ref[...] = _dot(feat, oW_ref[...]) + ob_ref[...]


def _pool_call(nf, batch_row, fcW, fcb, oW, ob):
    return pl.pallas_call(
        _pool_body,
        out_shape=jax.ShapeDtypeStruct((N_GRAPHS, 1), F32),
    )(nf, batch_row, fcW, fcb.reshape(1, C), oW, ob.reshape(1, 1))


def _sc_gather(table, idx, width, gch):
    epw = N_EDGES // SC_WORKERS

    @functools.partial(
        pl.kernel,
        mesh=plsc.VectorSubcoreMesh(**_SC_MESH),
        out_type=jax.ShapeDtypeStruct((N_EDGES, width), F32),
        scratch_types=[
            pltpu.VMEM((gch,), jnp.int32),
            pltpu.VMEM((gch, width), F32),
            pltpu.SemaphoreType.DMA,
        ],
    )
    def k(table_hbm, idx_hbm, out_hbm, idx_v, rows_v, sem):
        wid = lax.axis_index("s") * SC_CORES + lax.axis_index("c")
        base = wid * epw

        def body(i, carry):
            off = base + i * gch
            pltpu.sync_copy(idx_hbm.at[pl.ds(off, gch)], idx_v)
            pltpu.async_copy(table_hbm.at[idx_v], rows_v, sem).wait()
            pltpu.sync_copy(rows_v, out_hbm.at[pl.ds(off, gch)])
            return carry

        lax.fori_loop(0, epw // gch, body, 0)

    return k(table, idx)


def _sc_scatter_add(msg2_t, idx, zeros):
    rows = 16
    sch = 512
    nch = N_EDGES // sch

    @functools.partial(
        pl.kernel,
        mesh=plsc.VectorSubcoreMesh(**_SC_MESH),
        out_type=jax.ShapeDtypeStruct((HC, N_NODES), F32),
        scratch_types=[
            pltpu.VMEM((sch,), jnp.int32),
            pltpu.VMEM((sch,), jnp.int32),
            pltpu.VMEM((rows, sch), F32),
            pltpu.VMEM((rows, sch), F32),
            pltpu.VMEM((rows, N_NODES), F32),
            pltpu.SemaphoreType.DMA,
            pltpu.SemaphoreType.DMA,
            pltpu.SemaphoreType.DMA,
            pltpu.SemaphoreType.DMA,
        ],
        compiler_params=pltpu.CompilerParams(needs_layout_passes=False),
    )
    def k(msg_hbm, idx_hbm, zeros_hbm, agg_hbm,
          idx0, idx1, rv0, rv1, acc_v, is0, is1, ds0, ds1):
        wid = lax.axis_index("s") * SC_CORES + lax.axis_index("c")
        r0 = wid * rows
        pltpu.sync_copy(zeros_hbm, acc_v)
        lanes = lax.iota(jnp.int32, 16)
        idx_bufs = (idx0, idx1)
        row_bufs = (rv0, rv1)
        isems = (is0, is1)
        dsems = (ds0, ds1)

        def start(i, b):
            off = i * sch
            pltpu.async_copy(idx_hbm.at[pl.ds(off, sch)], idx_bufs[b],
                             isems[b])
            pltpu.async_copy(msg_hbm.at[pl.ds(r0, rows), pl.ds(off, sch)],
                             row_bufs[b], dsems[b])

        def wait(b):
            pltpu.make_async_copy(idx_hbm.at[pl.ds(0, sch)], idx_bufs[b],
                                  isems[b]).wait()
            pltpu.make_async_copy(msg_hbm.at[pl.ds(r0, rows), pl.ds(0, sch)],
                                  row_bufs[b], dsems[b]).wait()

        def compute(b):
            idx_v = idx_bufs[b]
            rows_v = row_bufs[b]

            def group(g, carry2):
                dstv = idx_v[pl.ds(g * 16, 16)]
                for f in range(rows):
                    vals = rows_v[f, pl.ds(g * 16, 16)]
                    plsc.addupdate_scatter(
                        acc_v, [jnp.full((16,), f, jnp.int32), dstv], vals)
                return carry2

            lax.fori_loop(0, sch // 16, group, 0)

        start(0, 0)

        def body2(i2, carry):
            i = i2 * 2

            @pl.when(i + 1 < nch)
            def _():
                start(i + 1, 1)

            wait(0)
            compute(0)

            @pl.when(i + 2 < nch)
            def _():
                start(i + 2, 0)

            @pl.when(i + 1 < nch)
            def _():
                wait(1)
                compute(1)

            return carry

        lax.fori_loop(0, (nch + 1) // 2, body2, 0)
        pltpu.sync_copy(acc_v, agg_hbm.at[pl.ds(r0, rows)])

    return k(msg2_t, idx, zeros)


def kernel(x, edge_index, edge_attr, batch, params):
    p = params
    src = edge_index[0].astype(jnp.int32)
    dst = edge_index[1].astype(jnp.int32)
    batch_row = batch.astype(jnp.int32).reshape(1, N_NODES)
    d2 = jnp.sum(edge_attr * edge_attr, axis=1, keepdims=True)
    zeros = jnp.zeros((16, N_NODES), F32)

    nf = _prep_call(x, p['atom_W'], p['atom_b'])
    ef = _ef_call(d2, p['rbf_W1'], p['rbf_b1'], p['rbf_W2'], p['rbf_b2'])

    for l in range(CONV_LAYERS):
        tdst, tsrc = _qkv_call(nf, p['Wq'][l], p['bq'][l], p['Wk'][l],
                               p['bk'][l], p['Wv'][l], p['bv'][l])
        gdst = _sc_gather(tdst, dst, 3 * HC, 64)
        gsrc = _sc_gather(tsrc, src, 2 * HC, 96)
        msg2 = _edge_call(gdst, gsrc, ef, p['We'][l], p['be'][l], p['Wm'][l],
                          p['bm'][l], p['Wmsg'][l], p['bmsg'][l],
                          p['ln_g'][l], p['ln_b'][l], p['lnm_g'][l],
                          p['lnm_b'][l])
        agg = _sc_scatter_add(msg2, dst, zeros)
        nf = _epi_call(agg, nf, p['Wc'][l], p['bc'][l], p['bn_g'][l],
                       p['bn_b'][l], p['Ws'][l], p['bs'][l])

    out = _pool_call(nf, batch_row, p['fc_W'], p['fc_b'], p['out_W'], p['out_b'])
    return out.reshape(N_GRAPHS)


# speedup vs baseline: 6.6924x; 1.0062x over previous
import functools

import jax
import jax.numpy as jnp
import numpy as np
from jax import lax
from jax.experimental import pallas as pl
from jax.experimental.pallas import tpu as pltpu
from jax.experimental.pallas import tpu_sc as plsc

N_NODES = 4096
N_EDGES = 49152
N_GRAPHS = 128
CONV_LAYERS = 5
ATOM_F = 92
RBF_BINS = 128
C = 128
H = 4
HC = H * C

F32 = jnp.float32

SC_CORES = 2
SC_SUBCORES = 16
SC_WORKERS = SC_CORES * SC_SUBCORES

_SC_MESH = dict(core_axis_name="c", subcore_axis_name="s")


def _dot(a, b, precision=lax.Precision.HIGHEST):
    return jnp.dot(a, b, preferred_element_type=F32, precision=precision)


def _dot_h(a, b):
    ah = a.astype(jnp.bfloat16)
    al = (a - ah.astype(F32)).astype(jnp.bfloat16)
    bh = b.astype(jnp.bfloat16)
    bl = (b - bh.astype(F32)).astype(jnp.bfloat16)
    d = functools.partial(jnp.dot, preferred_element_type=F32)
    return d(ah, bh) + (d(ah, bl) + d(al, bh))


def _prep_body(x_ref, aW_ref, ab_ref, o_ref):
    o_ref[...] = _dot(x_ref[...], aW_ref[...]) + ab_ref[...]


def _prep_call(x, aW, ab):
    return pl.pallas_call(
        _prep_body,
        out_shape=jax.ShapeDtypeStruct((N_NODES, C), F32),
    )(x, aW, ab.reshape(1, C))


def _ef_body(d2_ref, W1_ref, b1_ref, W2_ref, b2_ref, o_ref):
    d = jnp.sqrt(d2_ref[...])
    centers = lax.broadcasted_iota(jnp.int32, (1, RBF_BINS), 1).astype(F32) * (
        8.0 / (RBF_BINS - 1))
    gamma = 1.0 / (8.0 / (RBF_BINS - 1))
    rbf = jnp.exp(-gamma * (d - centers) ** 2)
    h = _dot(rbf, W1_ref[...]) + b1_ref[...]
    sp = jnp.maximum(h, 0.0) + jnp.log1p(jnp.exp(-jnp.abs(h)))
    o_ref[...] = _dot(sp, W2_ref[...]) + b2_ref[...]


def _ef_call(d2, W1, b1, W2, b2):
    B = 512
    grid = (N_EDGES // B,)
    return pl.pallas_call(
        _ef_body,
        grid=grid,
        in_specs=[
            pl.BlockSpec((B, 1), lambda i: (i, 0)),
            pl.BlockSpec((RBF_BINS, C), lambda i: (0, 0)),
            pl.BlockSpec((1, C), lambda i: (0, 0)),
            pl.BlockSpec((C, C), lambda i: (0, 0)),
            pl.BlockSpec((1, C), lambda i: (0, 0)),
        ],
        out_specs=pl.BlockSpec((B, C), lambda i: (i, 0)),
        out_shape=jax.ShapeDtypeStruct((N_EDGES, C), F32),
    )(d2, W1, b1.reshape(1, C), W2, b2.reshape(1, C))


def _qkv_body(nf_ref, Wq_ref, bq_ref, Wk_ref, bk_ref, Wv_ref, bv_ref,
              dst_ref, src_ref):
    nf = nf_ref[...]
    q = _dot(nf, Wq_ref[...]) + bq_ref[...]
    k = _dot(nf, Wk_ref[...]) + bk_ref[...]
    v = _dot(nf, Wv_ref[...]) + bv_ref[...]
    dst_ref[:, 0:HC] = q
    dst_ref[:, HC:2 * HC] = q * k
    dst_ref[:, 2 * HC:3 * HC] = v
    src_ref[:, 0:HC] = k
    src_ref[:, HC:2 * HC] = v


def _qkv_call(nf, Wq, bq, Wk, bk, Wv, bv):
    B = 1024
    grid = (N_NODES // B,)
    wspec = pl.BlockSpec((C, HC), lambda i: (0, 0))
    bspec = pl.BlockSpec((1, HC), lambda i: (0, 0))
    return pl.pallas_call(
        _qkv_body,
        grid=grid,
        in_specs=[pl.BlockSpec((B, C), lambda i: (i, 0)),
                  wspec, bspec, wspec, bspec, wspec, bspec],
        out_specs=(pl.BlockSpec((B, 3 * HC), lambda i: (i, 0)),
                   pl.BlockSpec((B, 2 * HC), lambda i: (i, 0))),
        out_shape=(jax.ShapeDtypeStruct((N_NODES, 3 * HC), F32),
                   jax.ShapeDtypeStruct((N_NODES, 2 * HC), F32)),
    )(nf, Wq, bq.reshape(1, HC), Wk, bk.reshape(1, HC), Wv, bv.reshape(1, HC))


def _edge_body(gdst_ref, gsrc_ref, ef_ref,
               We_ref, be_ref, Wm_ref, bm_ref, Wmsg_ref, bmsg_ref,
               lng_ref, lnb_ref, lnmg_ref, lnmb_ref, o_ref):
    scale = 1.0 / np.sqrt(3.0 * C)
    e_full = _dot_h(ef_ref[...], We_ref[...]) + be_ref[...]
    Wm = Wm_ref[...]
    Wmsg = Wmsg_ref[...]
    bm = bm_ref[...]
    lng = lng_ref[...]
    lnb = lnb_ref[...]
    for h in range(H):
        sl = slice(h * C, (h + 1) * C)
        q_h = gdst_ref[:, h * C:(h + 1) * C]
        g1_h = gdst_ref[:, HC + h * C:HC + (h + 1) * C]
        vd_h = gdst_ref[:, 2 * HC + h * C:2 * HC + (h + 1) * C]
        ks_h = gsrc_ref[:, h * C:(h + 1) * C]
        vs_h = gsrc_ref[:, HC + h * C:HC + (h + 1) * C]
        e_h = e_full[:, sl]
        a1 = g1_h * scale
        a2 = (q_h * ks_h) * scale
        a3 = (q_h * e_h) * scale
        m = (jnp.sum(a1, 1, keepdims=True) + jnp.sum(a2, 1, keepdims=True)
             + jnp.sum(a3, 1, keepdims=True)) * (1.0 / (3 * C))
        d1 = a1 - m
        d2 = a2 - m
        d3 = a3 - m
        var = (jnp.sum(d1 * d1, 1, keepdims=True)
               + jnp.sum(d2 * d2, 1, keepdims=True)
               + jnp.sum(d3 * d3, 1, keepdims=True)) * (1.0 / (3 * C))
        rstd = lax.rsqrt(var + 1e-5)
        s1 = jax.nn.sigmoid(d1 * rstd * lng[:, 0:C] + lnb[:, 0:C])
        s2 = jax.nn.sigmoid(d2 * rstd * lng[:, C:2 * C] + lnb[:, C:2 * C])
        s3 = jax.nn.sigmoid(d3 * rstd * lng[:, 2 * C:3 * C] + lnb[:, 2 * C:3 * C])
        sig = jnp.concatenate([s1, s2, s3], axis=1)
        msg = (_dot_h(vd_h, Wm[0:C, :]) + _dot_h(vs_h, Wm[C:2 * C, :])
               + _dot_h(e_h, Wm[2 * C:3 * C, :]) + bm)
        msg = msg * sig
        msg2 = _dot_h(msg, Wmsg) + bmsg_ref[...]
        m2 = jnp.mean(msg2, 1, keepdims=True)
        dv = msg2 - m2
        v2 = jnp.mean(dv * dv, 1, keepdims=True)
        res = dv * lax.rsqrt(v2 + 1e-5) * lnmg_ref[...] + lnmb_ref[...]
        o_ref[sl, :] = res.T


def _edge_call(gdst, gsrc, ef, We, be, Wm, bm, Wmsg, bmsg, lng, lnb, lnmg, lnmb):
    B = 512
    grid = (N_EDGES // B,)
    return pl.pallas_call(
        _edge_body,
        grid=grid,
        in_specs=[
            pl.BlockSpec((B, 3 * HC), lambda i: (i, 0)),
            pl.BlockSpec((B, 2 * HC), lambda i: (i, 0)),
            pl.BlockSpec((B, C), lambda i: (i, 0)),
            pl.BlockSpec((C, HC), lambda i: (0, 0)),
            pl.BlockSpec((1, HC), lambda i: (0, 0)),
            pl.BlockSpec((3 * C, 3 * C), lambda i: (0, 0)),
            pl.BlockSpec((1, 3 * C), lambda i: (0, 0)),
            pl.BlockSpec((3 * C, C), lambda i: (0, 0)),
            pl.BlockSpec((1, C), lambda i: (0, 0)),
            pl.BlockSpec((1, 3 * C), lambda i: (0, 0)),
            pl.BlockSpec((1, 3 * C), lambda i: (0, 0)),
            pl.BlockSpec((1, C), lambda i: (0, 0)),
            pl.BlockSpec((1, C), lambda i: (0, 0)),
        ],
        out_specs=pl.BlockSpec((HC, B), lambda i: (0, i)),
        out_shape=jax.ShapeDtypeStruct((HC, N_EDGES), F32),
    )(gdst, gsrc, ef, We, be.reshape(1, HC), Wm, bm.reshape(1, 3 * C),
      Wmsg, bmsg.reshape(1, C), lng.reshape(1, 3 * C), lnb.reshape(1, 3 * C),
      lnmg.reshape(1, C), lnmb.reshape(1, C))


def _epi_body(agg_ref, nf_ref, Wc_ref, bc_ref, bng_ref, bnb_ref,
              Ws_ref, bs_ref, o_ref):
    out = lax.dot_general(agg_ref[...], Wc_ref[...],
                          (((0,), (0,)), ((), ())),
                          preferred_element_type=F32,
                          precision=lax.Precision.HIGHEST) + bc_ref[...]
    mu = jnp.mean(out, 0, keepdims=True)
    dv = out - mu
    var = jnp.mean(dv * dv, 0, keepdims=True)
    out = dv * lax.rsqrt(var + 1e-5) * bng_ref[...] + bnb_ref[...]
    out = out * jax.nn.sigmoid(out)
    o_ref[...] = out + _dot(nf_ref[...], Ws_ref[...]) + bs_ref[...]


def _epi_call(agg, nf, Wc, bc, bng, bnb, Ws, bs):
    return pl.pallas_call(
        _epi_body,
        out_shape=jax.ShapeDtypeStruct((N_NODES, C), F32),
    )(agg, nf, Wc, bc.reshape(1, C), bng.reshape(1, C), bnb.reshape(1, C),
      Ws, bs.reshape(1, C))


def _pool_body(nf_ref, batch_ref, fcW_ref, fcb_ref, oW_ref, ob_ref, o_ref):
    ids = lax.broadcasted_iota(jnp.int32, (N_GRAPHS, 1), 0)
    oh = (batch_ref[...] == ids).astype(F32)
    counts = jnp.sum(oh, 1, keepdims=True)
    pooled = _dot(oh, nf_ref[...]) / jnp.maximum(counts, 1.0)
    feat = _dot(pooled, fcW_ref[...]) + fcb_ref[...]
    feat = feat * jax.nn.sigmoid(feat)
    o_ref[...] = _dot(feat, oW_ref[...]) + ob_ref[...]


def _pool_call(nf, batch_row, fcW, fcb, oW, ob):
    return pl.pallas_call(
        _pool_body,
        out_shape=jax.ShapeDtypeStruct((N_GRAPHS, 1), F32),
    )(nf, batch_row, fcW, fcb.reshape(1, C), oW, ob.reshape(1, 1))


def _sc_gather2(tdst, tsrc, dsti, srci):
    epw = N_EDGES // SC_WORKERS
    gch = 24
    nch = epw // gch

    @functools.partial(
        pl.kernel,
        mesh=plsc.VectorSubcoreMesh(**_SC_MESH),
        out_type=(jax.ShapeDtypeStruct((N_EDGES, 3 * HC), F32),
                  jax.ShapeDtypeStruct((N_EDGES, 2 * HC), F32)),
        scratch_types=[
            pltpu.VMEM((gch,), jnp.int32),
            pltpu.VMEM((gch,), jnp.int32),
            pltpu.VMEM((gch, 3 * HC), F32),
            pltpu.VMEM((gch, 3 * HC), F32),
            pltpu.VMEM((gch, 2 * HC), F32),
            pltpu.VMEM((gch, 2 * HC), F32),
            pltpu.SemaphoreType.DMA,
            pltpu.SemaphoreType.DMA,
            pltpu.SemaphoreType.DMA,
            pltpu.SemaphoreType.DMA,
            pltpu.SemaphoreType.DMA,
            pltpu.SemaphoreType.DMA,
        ],
    )
    def k(tdst_hbm, tsrc_hbm, dsti_hbm, srci_hbm, gdst_hbm, gsrc_hbm,
          ib0, ib1, rd0, rd1, rs0, rs1, is0, is1, gs0, gs1, ws0, ws1):
        wid = lax.axis_index("s") * SC_CORES + lax.axis_index("c")
        base = wid * epw

        def run_phase(table, idxh, outh, rb):
            ib = (ib0, ib1)
            isems = (is0, is1)
            gsems = (gs0, gs1)
            wsems = (ws0, ws1)

            def idx_start(i, b):
                pltpu.async_copy(idxh.at[pl.ds(base + i * gch, gch)], ib[b],
                                 isems[b])

            def idx_wait(b):
                pltpu.make_async_copy(idxh.at[pl.ds(0, gch)], ib[b],
                                      isems[b]).wait()

            def g_start(b):
                pltpu.async_copy(table.at[ib[b]], rb[b], gsems[b])

            def g_wait(b):
                pltpu.make_async_copy(table.at[ib[b]], rb[b], gsems[b]).wait()

            def w_start(i, b):
                pltpu.async_copy(rb[b], outh.at[pl.ds(base + i * gch, gch)],
                                 wsems[b])

            def w_wait(b):
                pltpu.make_async_copy(rb[b], outh.at[pl.ds(0, gch)],
                                      wsems[b]).wait()

            idx_start(0, 0)
            idx_wait(0)
            g_start(0)
            idx_start(1, 1)

            def body(i2, carry):
                i = 2 * i2
                g_wait(0)
                w_start(i, 0)
                idx_wait(1)

                @pl.when(i2 > 0)
                def _():
                    w_wait(1)

                g_start(1)

                @pl.when(i + 2 < nch)
                def _():
                    idx_start(i + 2, 0)

                g_wait(1)
                w_start(i + 1, 1)

                @pl.when(i + 2 < nch)
                def _():
                    idx_wait(0)
                    w_wait(0)
                    g_start(0)

                @pl.when(i + 3 < nch)
                def _():
                    idx_start(i + 3, 1)

                return carry

            lax.fori_loop(0, nch // 2, body, 0)
            w_wait(0)
            w_wait(1)

        run_phase(tdst_hbm, dsti_hbm, gdst_hbm, (rd0, rd1))
        run_phase(tsrc_hbm, srci_hbm, gsrc_hbm, (rs0, rs1))

    return k(tdst, tsrc, dsti, srci)


def _sc_scatter_add(msg2_t, idx, zeros):
    rows = 16
    sch = 512
    nch = N_EDGES // sch

    @functools.partial(
        pl.kernel,
        mesh=plsc.VectorSubcoreMesh(**_SC_MESH),
        out_type=jax.ShapeDtypeStruct((HC, N_NODES), F32),
        scratch_types=[
            pltpu.VMEM((sch,), jnp.int32),
            pltpu.VMEM((sch,), jnp.int32),
            pltpu.VMEM((rows, sch), F32),
            pltpu.VMEM((rows, sch), F32),
            pltpu.VMEM((rows, N_NODES), F32),
            pltpu.SemaphoreType.DMA,
            pltpu.SemaphoreType.DMA,
            pltpu.SemaphoreType.DMA,
            pltpu.SemaphoreType.DMA,
        ],
        compiler_params=pltpu.CompilerParams(needs_layout_passes=False),
    )
    def k(msg_hbm, idx_hbm, zeros_hbm, agg_hbm,
          idx0, idx1, rv0, rv1, acc_v, is0, is1, ds0, ds1):
        wid = lax.axis_index("s") * SC_CORES + lax.axis_index("c")
        r0 = wid * rows
        pltpu.sync_copy(zeros_hbm, acc_v)
        lanes = lax.iota(jnp.int32, 16)
        idx_bufs = (idx0, idx1)
        row_bufs = (rv0, rv1)
        isems = (is0, is1)
        dsems = (ds0, ds1)

        def start(i, b):
            off = i * sch
            pltpu.async_copy(idx_hbm.at[pl.ds(off, sch)], idx_bufs[b],
                             isems[b])
            pltpu.async_copy(msg_hbm.at[pl.ds(r0, rows), pl.ds(off, sch)],
                             row_bufs[b], dsems[b])

        def wait(b):
            pltpu.make_async_copy(idx_hbm.at[pl.ds(0, sch)], idx_bufs[b],
                                  isems[b]).wait()
            pltpu.make_async_copy(msg_hbm.at[pl.ds(r0, rows), pl.ds(0, sch)],
                                  row_bufs[b], dsems[b]).wait()

        def compute(b):
            idx_v = idx_bufs[b]
            rows_v = row_bufs[b]

            def group(g, carry2):
                dstv = idx_v[pl.ds(g * 16, 16)]
                for f in range(rows):
                    vals = rows_v[f, pl.ds(g * 16, 16)]
                    plsc.addupdate_scatter(
                        acc_v, [jnp.full((16,), f, jnp.int32), dstv], vals)
                return carry2

            lax.fori_loop(0, sch // 16, group, 0)

        start(0, 0)

        def body2(i2, carry):
            i = i2 * 2

            @pl.when(i + 1 < nch)
            def _():
                start(i + 1, 1)

            wait(0)
            compute(0)

            @pl.when(i + 2 < nch)
            def _():
                start(i + 2, 0)

            @pl.when(i + 1 < nch)
            def _():
                wait(1)
                compute(1)

            return carry

        lax.fori_loop(0, (nch + 1) // 2, body2, 0)
        pltpu.sync_copy(acc_v, agg_hbm.at[pl.ds(r0, rows)])

    return k(msg2_t, idx, zeros)


def kernel(x, edge_index, edge_attr, batch, params):
    p = params
    src = edge_index[0].astype(jnp.int32)
    dst = edge_index[1].astype(jnp.int32)
    batch_row = batch.astype(jnp.int32).reshape(1, N_NODES)
    d2 = jnp.sum(edge_attr * edge_attr, axis=1, keepdims=True)
    zeros = jnp.zeros((16, N_NODES), F32)

    nf = _prep_call(x, p['atom_W'], p['atom_b'])
    ef = _ef_call(d2, p['rbf_W1'], p['rbf_b1'], p['rbf_W2'], p['rbf_b2'])

    for l in range(CONV_LAYERS):
        tdst, tsrc = _qkv_call(nf, p['Wq'][l], p['bq'][l], p['Wk'][l],
                               p['bk'][l], p['Wv'][l], p['bv'][l])
        gdst, gsrc = _sc_gather2(tdst, tsrc, dst, src)
        msg2 = _edge_call(gdst, gsrc, ef, p['We'][l], p['be'][l], p['Wm'][l],
                          p['bm'][l], p['Wmsg'][l], p['bmsg'][l],
                          p['ln_g'][l], p['ln_b'][l], p['lnm_g'][l],
                          p['lnm_b'][l])
        agg = _sc_scatter_add(msg2, dst, zeros)
        nf = _epi_call(agg, nf, p['Wc'][l], p['bc'][l], p['bn_g'][l],
                       p['bn_b'][l], p['Ws'][l], p['bs'][l])

    out = _pool_call(nf, batch_row, p['fc_W'], p['fc_b'], p['out_W'], p['out_b'])
    return out.reshape(N_GRAPHS)


# speedup vs baseline: 8.4091x; 1.2565x over previous
import functools

import jax
import jax.numpy as jnp
import numpy as np
from jax import lax
from jax.experimental import pallas as pl
from jax.experimental.pallas import tpu as pltpu
from jax.experimental.pallas import tpu_sc as plsc

N_NODES = 4096
N_EDGES = 49152
N_GRAPHS = 128
CONV_LAYERS = 5
ATOM_F = 92
RBF_BINS = 128
C = 128
H = 4
HC = H * C

F32 = jnp.float32

SC_CORES = 2
SC_SUBCORES = 16
SC_WORKERS = SC_CORES * SC_SUBCORES

_SC_MESH = dict(core_axis_name="c", subcore_axis_name="s")


def _dot(a, b, precision=lax.Precision.HIGHEST):
    return jnp.dot(a, b, preferred_element_type=F32, precision=precision)


def _dot_h(a, b):
    ah = a.astype(jnp.bfloat16)
    al = (a - ah.astype(F32)).astype(jnp.bfloat16)
    bh = b.astype(jnp.bfloat16)
    bl = (b - bh.astype(F32)).astype(jnp.bfloat16)
    d = functools.partial(jnp.dot, preferred_element_type=F32)
    return d(ah, bh) + (d(ah, bl) + d(al, bh))


def _prep_body(x_ref, aW_ref, ab_ref, o_ref):
    o_ref[...] = _dot(x_ref[...], aW_ref[...]) + ab_ref[...]


def _prep_call(x, aW, ab):
    return pl.pallas_call(
        _prep_body,
        out_shape=jax.ShapeDtypeStruct((N_NODES, C), F32),
    )(x, aW, ab.reshape(1, C))


def _ef_body(d2_ref, W1_ref, b1_ref, W2_ref, b2_ref, o_ref):
    d = jnp.sqrt(d2_ref[...])
    centers = lax.broadcasted_iota(jnp.int32, (1, RBF_BINS), 1).astype(F32) * (
        8.0 / (RBF_BINS - 1))
    gamma = 1.0 / (8.0 / (RBF_BINS - 1))
    rbf = jnp.exp(-gamma * (d - centers) ** 2)
    h = _dot(rbf, W1_ref[...]) + b1_ref[...]
    sp = jnp.maximum(h, 0.0) + jnp.log1p(jnp.exp(-jnp.abs(h)))
    o_ref[...] = _dot(sp, W2_ref[...]) + b2_ref[...]


def _ef_call(d2, W1, b1, W2, b2):
    B = 512
    grid = (N_EDGES // B,)
    return pl.pallas_call(
        _ef_body,
        grid=grid,
        in_specs=[
            pl.BlockSpec((B, 1), lambda i: (i, 0)),
            pl.BlockSpec((RBF_BINS, C), lambda i: (0, 0)),
            pl.BlockSpec((1, C), lambda i: (0, 0)),
            pl.BlockSpec((C, C), lambda i: (0, 0)),
            pl.BlockSpec((1, C), lambda i: (0, 0)),
        ],
        out_specs=pl.BlockSpec((B, C), lambda i: (i, 0)),
        out_shape=jax.ShapeDtypeStruct((N_EDGES, C), F32),
    )(d2, W1, b1.reshape(1, C), W2, b2.reshape(1, C))


def _qkv_body(nf_ref, Wq_ref, bq_ref, Wk_ref, bk_ref, Wv_ref, bv_ref,
              dst_ref, src_ref):
    nf = nf_ref[...]
    q = _dot(nf, Wq_ref[...]) + bq_ref[...]
    k = _dot(nf, Wk_ref[...]) + bk_ref[...]
    v = _dot(nf, Wv_ref[...]) + bv_ref[...]
    dst_ref[:, 0:HC] = q
    dst_ref[:, HC:2 * HC] = q * k
    dst_ref[:, 2 * HC:3 * HC] = v
    src_ref[:, 0:HC] = k
    src_ref[:, HC:2 * HC] = v


def _qkv_call(nf, Wq, bq, Wk, bk, Wv, bv):
    B = 1024
    grid = (N_NODES // B,)
    wspec = pl.BlockSpec((C, HC), lambda i: (0, 0))
    bspec = pl.BlockSpec((1, HC), lambda i: (0, 0))
    return pl.pallas_call(
        _qkv_body,
        grid=grid,
        in_specs=[pl.BlockSpec((B, C), lambda i: (i, 0)),
                  wspec, bspec, wspec, bspec, wspec, bspec],
        out_specs=(pl.BlockSpec((B, 3 * HC), lambda i: (i, 0)),
                   pl.BlockSpec((B, 2 * HC), lambda i: (i, 0))),
        out_shape=(jax.ShapeDtypeStruct((N_NODES, 3 * HC), F32),
                   jax.ShapeDtypeStruct((N_NODES, 2 * HC), F32)),
    )(nf, Wq, bq.reshape(1, HC), Wk, bk.reshape(1, HC), Wv, bv.reshape(1, HC))


def _edge_body(gdst_ref, gsrc_ref, ef_ref,
               We_ref, be_ref, Wm_ref, bm_ref, Wmsg_ref, bmsg_ref,
               lng_ref, lnb_ref, lnmg_ref, lnmb_ref, o_ref):
    scale = 1.0 / np.sqrt(3.0 * C)
    e_full = _dot_h(ef_ref[...], We_ref[...]) + be_ref[...]
    Wm = Wm_ref[...]
    Wmsg = Wmsg_ref[...]
    bm = bm_ref[...]
    lng = lng_ref[...]
    lnb = lnb_ref[...]
    for h in range(H):
        sl = slice(h * C, (h + 1) * C)
        q_h = gdst_ref[:, h * C:(h + 1) * C]
        g1_h = gdst_ref[:, HC + h * C:HC + (h + 1) * C]
        vd_h = gdst_ref[:, 2 * HC + h * C:2 * HC + (h + 1) * C]
        ks_h = gsrc_ref[:, h * C:(h + 1) * C]
        vs_h = gsrc_ref[:, HC + h * C:HC + (h + 1) * C]
        e_h = e_full[:, sl]
        a1 = g1_h * scale
        a2 = (q_h * ks_h) * scale
        a3 = (q_h * e_h) * scale
        m = (jnp.sum(a1, 1, keepdims=True) + jnp.sum(a2, 1, keepdims=True)
             + jnp.sum(a3, 1, keepdims=True)) * (1.0 / (3 * C))
        d1 = a1 - m
        d2 = a2 - m
        d3 = a3 - m
        var = (jnp.sum(d1 * d1, 1, keepdims=True)
               + jnp.sum(d2 * d2, 1, keepdims=True)
               + jnp.sum(d3 * d3, 1, keepdims=True)) * (1.0 / (3 * C))
        rstd = lax.rsqrt(var + 1e-5)
        s1 = jax.nn.sigmoid(d1 * rstd * lng[:, 0:C] + lnb[:, 0:C])
        s2 = jax.nn.sigmoid(d2 * rstd * lng[:, C:2 * C] + lnb[:, C:2 * C])
        s3 = jax.nn.sigmoid(d3 * rstd * lng[:, 2 * C:3 * C] + lnb[:, 2 * C:3 * C])
        sig = jnp.concatenate([s1, s2, s3], axis=1)
        msg = (_dot_h(vd_h, Wm[0:C, :]) + _dot_h(vs_h, Wm[C:2 * C, :])
               + _dot_h(e_h, Wm[2 * C:3 * C, :]) + bm)
        msg = msg * sig
        msg2 = _dot_h(msg, Wmsg) + bmsg_ref[...]
        m2 = jnp.mean(msg2, 1, keepdims=True)
        dv = msg2 - m2
        v2 = jnp.mean(dv * dv, 1, keepdims=True)
        res = dv * lax.rsqrt(v2 + 1e-5) * lnmg_ref[...] + lnmb_ref[...]
        o_ref[sl, :] = res.T


def _edge_call(gdst, gsrc, ef, e0, n_edges,
               We, be, Wm, bm, Wmsg, bmsg, lng, lnb, lnmg, lnmb):
    B = 512
    grid = (n_edges // B,)
    e0b = e0 // B
    return pl.pallas_call(
        _edge_body,
        grid=grid,
        in_specs=[
            pl.BlockSpec((B, 3 * HC), lambda i: (i, 0)),
            pl.BlockSpec((B, 2 * HC), lambda i: (i, 0)),
            pl.BlockSpec((B, C), lambda i: (i + e0b, 0)),
            pl.BlockSpec((C, HC), lambda i: (0, 0)),
            pl.BlockSpec((1, HC), lambda i: (0, 0)),
            pl.BlockSpec((3 * C, 3 * C), lambda i: (0, 0)),
            pl.BlockSpec((1, 3 * C), lambda i: (0, 0)),
            pl.BlockSpec((3 * C, C), lambda i: (0, 0)),
            pl.BlockSpec((1, C), lambda i: (0, 0)),
            pl.BlockSpec((1, 3 * C), lambda i: (0, 0)),
            pl.BlockSpec((1, 3 * C), lambda i: (0, 0)),
            pl.BlockSpec((1, C), lambda i: (0, 0)),
            pl.BlockSpec((1, C), lambda i: (0, 0)),
        ],
        out_specs=pl.BlockSpec((HC, B), lambda i: (0, i)),
        out_shape=jax.ShapeDtypeStruct((HC, n_edges), F32),
    )(gdst, gsrc, ef, We, be.reshape(1, HC), Wm, bm.reshape(1, 3 * C),
      Wmsg, bmsg.reshape(1, C), lng.reshape(1, 3 * C), lnb.reshape(1, 3 * C),
      lnmg.reshape(1, C), lnmb.reshape(1, C))


def _epi_body(agg1_ref, agg2_ref, nf_ref, Wc_ref, bc_ref, bng_ref, bnb_ref,
              Ws_ref, bs_ref, o_ref):
    agg = agg1_ref[...] + agg2_ref[...]
    out = lax.dot_general(agg, Wc_ref[...],
                          (((0,), (0,)), ((), ())),
                          preferred_element_type=F32,
                          precision=lax.Precision.HIGHEST) + bc_ref[...]
    mu = jnp.mean(out, 0, keepdims=True)
    dv = out - mu
    var = jnp.mean(dv * dv, 0, keepdims=True)
    out = dv * lax.rsqrt(var + 1e-5) * bng_ref[...] + bnb_ref[...]
    out = out * jax.nn.sigmoid(out)
    o_ref[...] = out + _dot(nf_ref[...], Ws_ref[...]) + bs_ref[...]


def _epi_call(agg1, agg2, nf, Wc, bc, bng, bnb, Ws, bs):
    return pl.pallas_call(
        _epi_body,
        out_shape=jax.ShapeDtypeStruct((N_NODES, C), F32),
    )(agg1, agg2, nf, Wc, bc.reshape(1, C), bng.reshape(1, C),
      bnb.reshape(1, C), Ws, bs.reshape(1, C))


def _pool_body(nf_ref, batch_ref, fcW_ref, fcb_ref, oW_ref, ob_ref, o_ref):
    ids = lax.broadcasted_iota(jnp.int32, (N_GRAPHS, 1), 0)
    oh = (batch_ref[...] == ids).astype(F32)
    counts = jnp.sum(oh, 1, keepdims=True)
    pooled = _dot(oh, nf_ref[...]) / jnp.maximum(counts, 1.0)
    feat = _dot(pooled, fcW_ref[...]) + fcb_ref[...]
    feat = feat * jax.nn.sigmoid(feat)
    o_ref[...] = _dot(feat, oW_ref[...]) + ob_ref[...]


def _pool_call(nf, batch_row, fcW, fcb, oW, ob):
    return pl.pallas_call(
        _pool_body,
        out_shape=jax.ShapeDtypeStruct((N_GRAPHS, 1), F32),
    )(nf, batch_row, fcW, fcb.reshape(1, C), oW, ob.reshape(1, 1))


def _sc_gather2(tdst, tsrc, dsti, srci, n_edges):
    epw = n_edges // SC_WORKERS
    gch = 24
    nch = epw // gch

    @functools.partial(
        pl.kernel,
        mesh=plsc.VectorSubcoreMesh(**_SC_MESH),
        out_type=(jax.ShapeDtypeStruct((n_edges, 3 * HC), F32),
                  jax.ShapeDtypeStruct((n_edges, 2 * HC), F32)),
        scratch_types=[
            pltpu.VMEM((gch,), jnp.int32),
            pltpu.VMEM((gch,), jnp.int32),
            pltpu.VMEM((gch, 3 * HC), F32),
            pltpu.VMEM((gch, 3 * HC), F32),
            pltpu.VMEM((gch, 2 * HC), F32),
            pltpu.VMEM((gch, 2 * HC), F32),
            pltpu.SemaphoreType.DMA,
            pltpu.SemaphoreType.DMA,
            pltpu.SemaphoreType.DMA,
            pltpu.SemaphoreType.DMA,
            pltpu.SemaphoreType.DMA,
            pltpu.SemaphoreType.DMA,
        ],
    )
    def k(tdst_hbm, tsrc_hbm, dsti_hbm, srci_hbm, gdst_hbm, gsrc_hbm,
          ib0, ib1, rd0, rd1, rs0, rs1, is0, is1, gs0, gs1, ws0, ws1):
        wid = lax.axis_index("s") * SC_CORES + lax.axis_index("c")
        base = wid * epw

        def run_phase(table, idxh, outh, rb):
            ib = (ib0, ib1)
            isems = (is0, is1)
            gsems = (gs0, gs1)
            wsems = (ws0, ws1)

            def idx_start(i, b):
                pltpu.async_copy(idxh.at[pl.ds(base + i * gch, gch)], ib[b],
                                 isems[b])

            def idx_wait(b):
                pltpu.make_async_copy(idxh.at[pl.ds(0, gch)], ib[b],
                                      isems[b]).wait()

            def g_start(b):
                pltpu.async_copy(table.at[ib[b]], rb[b], gsems[b])

            def g_wait(b):
                pltpu.make_async_copy(table.at[ib[b]], rb[b], gsems[b]).wait()

            def w_start(i, b):
                pltpu.async_copy(rb[b], outh.at[pl.ds(base + i * gch, gch)],
                                 wsems[b])

            def w_wait(b):
                pltpu.make_async_copy(rb[b], outh.at[pl.ds(0, gch)],
                                      wsems[b]).wait()

            idx_start(0, 0)
            idx_wait(0)
            g_start(0)
            idx_start(1, 1)

            def body(i2, carry):
                i = 2 * i2
                g_wait(0)
                w_start(i, 0)
                idx_wait(1)

                @pl.when(i2 > 0)
                def _():
                    w_wait(1)

                g_start(1)

                @pl.when(i + 2 < nch)
                def _():
                    idx_start(i + 2, 0)

                g_wait(1)
                w_start(i + 1, 1)

                @pl.when(i + 2 < nch)
                def _():
                    idx_wait(0)
                    w_wait(0)
                    g_start(0)

                @pl.when(i + 3 < nch)
                def _():
                    idx_start(i + 3, 1)

                return carry

            lax.fori_loop(0, nch // 2, body, 0)
            w_wait(0)
            w_wait(1)

        run_phase(tdst_hbm, dsti_hbm, gdst_hbm, (rd0, rd1))
        run_phase(tsrc_hbm, srci_hbm, gsrc_hbm, (rs0, rs1))

    return k(tdst, tsrc, dsti, srci)


def _sc_scatter_add(msg2_t, idx, zeros, n_edges):
    rows = 16
    sch = 512
    nch = n_edges // sch

    @functools.partial(
        pl.kernel,
        mesh=plsc.VectorSubcoreMesh(**_SC_MESH),
        out_type=jax.ShapeDtypeStruct((HC, N_NODES), F32),
        scratch_types=[
            pltpu.VMEM((sch,), jnp.int32),
            pltpu.VMEM((sch,), jnp.int32),
            pltpu.VMEM((rows, sch), F32),
            pltpu.VMEM((rows, sch), F32),
            pltpu.VMEM((rows, N_NODES), F32),
            pltpu.SemaphoreType.DMA,
            pltpu.SemaphoreType.DMA,
            pltpu.SemaphoreType.DMA,
            pltpu.SemaphoreType.DMA,
        ],
        compiler_params=pltpu.CompilerParams(needs_layout_passes=False),
    )
    def k(msg_hbm, idx_hbm, zeros_hbm, agg_hbm,
          idx0, idx1, rv0, rv1, acc_v, is0, is1, ds0, ds1):
        wid = lax.axis_index("s") * SC_CORES + lax.axis_index("c")
        r0 = wid * rows
        pltpu.sync_copy(zeros_hbm, acc_v)
        lanes = lax.iota(jnp.int32, 16)
        idx_bufs = (idx0, idx1)
        row_bufs = (rv0, rv1)
        isems = (is0, is1)
        dsems = (ds0, ds1)

        def start(i, b):
            off = i * sch
            pltpu.async_copy(idx_hbm.at[pl.ds(off, sch)], idx_bufs[b],
                             isems[b])
            pltpu.async_copy(msg_hbm.at[pl.ds(r0, rows), pl.ds(off, sch)],
                             row_bufs[b], dsems[b])

        def wait(b):
            pltpu.make_async_copy(idx_hbm.at[pl.ds(0, sch)], idx_bufs[b],
                                  isems[b]).wait()
            pltpu.make_async_copy(msg_hbm.at[pl.ds(r0, rows), pl.ds(0, sch)],
                                  row_bufs[b], dsems[b]).wait()

        def compute(b):
            idx_v = idx_bufs[b]
            rows_v = row_bufs[b]

            def group(g, carry2):
                dstv = idx_v[pl.ds(g * 16, 16)]
                for f in range(rows):
                    vals = rows_v[f, pl.ds(g * 16, 16)]
                    plsc.addupdate_scatter(
                        acc_v, [jnp.full((16,), f, jnp.int32), dstv], vals)
                return carry2

            lax.fori_loop(0, sch // 16, group, 0)

        start(0, 0)

        def body2(i2, carry):
            i = i2 * 2

            @pl.when(i + 1 < nch)
            def _():
                start(i + 1, 1)

            wait(0)
            compute(0)

            @pl.when(i + 2 < nch)
            def _():
                start(i + 2, 0)

            @pl.when(i + 1 < nch)
            def _():
                wait(1)
                compute(1)

            return carry

        lax.fori_loop(0, (nch + 1) // 2, body2, 0)
        pltpu.sync_copy(acc_v, agg_hbm.at[pl.ds(r0, rows)])

    return k(msg2_t, idx, zeros)


def kernel(x, edge_index, edge_attr, batch, params):
    p = params
    src = edge_index[0].astype(jnp.int32)
    dst = edge_index[1].astype(jnp.int32)
    batch_row = batch.astype(jnp.int32).reshape(1, N_NODES)
    d2 = jnp.sum(edge_attr * edge_attr, axis=1, keepdims=True)
    zeros = jnp.zeros((16, N_NODES), F32)

    nf = _prep_call(x, p['atom_W'], p['atom_b'])
    ef = _ef_call(d2, p['rbf_W1'], p['rbf_b1'], p['rbf_W2'], p['rbf_b2'])

    E2 = N_EDGES // 2
    halves = ((dst[:E2], src[:E2], 0), (dst[E2:], src[E2:], E2))
    for l in range(CONV_LAYERS):
        tdst, tsrc = _qkv_call(nf, p['Wq'][l], p['bq'][l], p['Wk'][l],
                               p['bk'][l], p['Wv'][l], p['bv'][l])
        aggs = []
        gathered = [_sc_gather2(tdst, tsrc, d_h, s_h, E2)
                    for d_h, s_h, _ in halves]
        for (d_h, s_h, e0), (gd, gs) in zip(halves, gathered):
            m_h = _edge_call(gd, gs, ef, e0, E2, p['We'][l], p['be'][l],
                             p['Wm'][l], p['bm'][l], p['Wmsg'][l],
                             p['bmsg'][l], p['ln_g'][l], p['ln_b'][l],
                             p['lnm_g'][l], p['lnm_b'][l])
            aggs.append(_sc_scatter_add(m_h, d_h, zeros, E2))
        nf = _epi_call(aggs[0], aggs[1], nf, p['Wc'][l], p['bc'][l],
                       p['bn_g'][l], p['bn_b'][l], p['Ws'][l], p['bs'][l])

    out = _pool_call(nf, batch_row, p['fc_W'], p['fc_b'], p['out_W'], p['out_b'])
    return out.reshape(N_GRAPHS)


# speedup vs baseline: 9.1635x; 1.0897x over previous
import functools

import jax
import jax.numpy as jnp
import numpy as np
from jax import lax
from jax.experimental import pallas as pl
from jax.experimental.pallas import tpu as pltpu
from jax.experimental.pallas import tpu_sc as plsc

N_NODES = 4096
N_EDGES = 49152
N_GRAPHS = 128
CONV_LAYERS = 5
ATOM_F = 92
RBF_BINS = 128
C = 128
H = 4
HC = H * C

F32 = jnp.float32

SC_CORES = 2
SC_SUBCORES = 16
SC_WORKERS = SC_CORES * SC_SUBCORES

_SC_MESH = dict(core_axis_name="c", subcore_axis_name="s")


def _dot(a, b, precision=lax.Precision.HIGHEST):
    return jnp.dot(a, b, preferred_element_type=F32, precision=precision)


def _dot_h(a, b):
    ah = a.astype(jnp.bfloat16)
    al = (a - ah.astype(F32)).astype(jnp.bfloat16)
    bh = b.astype(jnp.bfloat16)
    bl = (b - bh.astype(F32)).astype(jnp.bfloat16)
    d = functools.partial(jnp.dot, preferred_element_type=F32)
    return d(ah, bh) + (d(ah, bl) + d(al, bh))


def _prep_body(x_ref, aW_ref, ab_ref, o_ref):
    o_ref[...] = _dot(x_ref[...], aW_ref[...]) + ab_ref[...]


def _prep_call(x, aW, ab):
    return pl.pallas_call(
        _prep_body,
        out_shape=jax.ShapeDtypeStruct((N_NODES, C), F32),
    )(x, aW, ab.reshape(1, C))


def _ef_body(d2_ref, W1_ref, b1_ref, W2_ref, b2_ref, o_ref):
    d = jnp.sqrt(d2_ref[...])
    centers = lax.broadcasted_iota(jnp.int32, (1, RBF_BINS), 1).astype(F32) * (
        8.0 / (RBF_BINS - 1))
    gamma = 1.0 / (8.0 / (RBF_BINS - 1))
    rbf = jnp.exp(-gamma * (d - centers) ** 2)
    h = _dot(rbf, W1_ref[...]) + b1_ref[...]
    sp = jnp.maximum(h, 0.0) + jnp.log1p(jnp.exp(-jnp.abs(h)))
    o_ref[...] = _dot(sp, W2_ref[...]) + b2_ref[...]


def _ef_call(d2, W1, b1, W2, b2):
    B = 512
    grid = (N_EDGES // B,)
    return pl.pallas_call(
        _ef_body,
        grid=grid,
        in_specs=[
            pl.BlockSpec((B, 1), lambda i: (i, 0)),
            pl.BlockSpec((RBF_BINS, C), lambda i: (0, 0)),
            pl.BlockSpec((1, C), lambda i: (0, 0)),
            pl.BlockSpec((C, C), lambda i: (0, 0)),
            pl.BlockSpec((1, C), lambda i: (0, 0)),
        ],
        out_specs=pl.BlockSpec((B, C), lambda i: (i, 0)),
        out_shape=jax.ShapeDtypeStruct((N_EDGES, C), F32),
    )(d2, W1, b1.reshape(1, C), W2, b2.reshape(1, C))


def _qkv_body(nf_ref, Wq_ref, bq_ref, Wk_ref, bk_ref, Wv_ref, bv_ref,
              dst_ref, src_ref):
    nf = nf_ref[...]
    q = _dot(nf, Wq_ref[...]) + bq_ref[...]
    k = _dot(nf, Wk_ref[...]) + bk_ref[...]
    v = _dot(nf, Wv_ref[...]) + bv_ref[...]
    dst_ref[:, 0:HC] = q
    dst_ref[:, HC:2 * HC] = q * k
    dst_ref[:, 2 * HC:3 * HC] = v
    src_ref[:, 0:HC] = k
    src_ref[:, HC:2 * HC] = v


def _qkv_call(nf, Wq, bq, Wk, bk, Wv, bv):
    B = 1024
    grid = (N_NODES // B,)
    wspec = pl.BlockSpec((C, HC), lambda i: (0, 0))
    bspec = pl.BlockSpec((1, HC), lambda i: (0, 0))
    return pl.pallas_call(
        _qkv_body,
        grid=grid,
        in_specs=[pl.BlockSpec((B, C), lambda i: (i, 0)),
                  wspec, bspec, wspec, bspec, wspec, bspec],
        out_specs=(pl.BlockSpec((B, 3 * HC), lambda i: (i, 0)),
                   pl.BlockSpec((B, 2 * HC), lambda i: (i, 0))),
        out_shape=(jax.ShapeDtypeStruct((N_NODES, 3 * HC), F32),
                   jax.ShapeDtypeStruct((N_NODES, 2 * HC), F32)),
    )(nf, Wq, bq.reshape(1, HC), Wk, bk.reshape(1, HC), Wv, bv.reshape(1, HC))


def _edge_body(gdst_ref, gsrc_ref, ef_ref,
               We_ref, be_ref, Wm_ref, bm_ref, Wmsg_ref, bmsg_ref,
               lng_ref, lnb_ref, lnmg_ref, lnmb_ref, o_ref):
    scale = 1.0 / np.sqrt(3.0 * C)
    e_full = _dot_h(ef_ref[...], We_ref[...]) + be_ref[...]
    Wm = Wm_ref[...]
    Wmsg = Wmsg_ref[...]
    bm = bm_ref[...]
    lng = lng_ref[...]
    lnb = lnb_ref[...]
    for h in range(H):
        sl = slice(h * C, (h + 1) * C)
        q_h = gdst_ref[:, h * C:(h + 1) * C]
        g1_h = gdst_ref[:, HC + h * C:HC + (h + 1) * C]
        vd_h = gdst_ref[:, 2 * HC + h * C:2 * HC + (h + 1) * C]
        ks_h = gsrc_ref[:, h * C:(h + 1) * C]
        vs_h = gsrc_ref[:, HC + h * C:HC + (h + 1) * C]
        e_h = e_full[:, sl]
        a1 = g1_h * scale
        a2 = (q_h * ks_h) * scale
        a3 = (q_h * e_h) * scale
        m = (jnp.sum(a1, 1, keepdims=True) + jnp.sum(a2, 1, keepdims=True)
             + jnp.sum(a3, 1, keepdims=True)) * (1.0 / (3 * C))
        d1 = a1 - m
        d2 = a2 - m
        d3 = a3 - m
        var = (jnp.sum(d1 * d1, 1, keepdims=True)
               + jnp.sum(d2 * d2, 1, keepdims=True)
               + jnp.sum(d3 * d3, 1, keepdims=True)) * (1.0 / (3 * C))
        rstd = lax.rsqrt(var + 1e-5)
        s1 = jax.nn.sigmoid(d1 * rstd * lng[:, 0:C] + lnb[:, 0:C])
        s2 = jax.nn.sigmoid(d2 * rstd * lng[:, C:2 * C] + lnb[:, C:2 * C])
        s3 = jax.nn.sigmoid(d3 * rstd * lng[:, 2 * C:3 * C] + lnb[:, 2 * C:3 * C])
        sig = jnp.concatenate([s1, s2, s3], axis=1)
        msg = (_dot_h(vd_h, Wm[0:C, :]) + _dot_h(vs_h, Wm[C:2 * C, :])
               + _dot_h(e_h, Wm[2 * C:3 * C, :]) + bm)
        msg = msg * sig
        msg2 = _dot_h(msg, Wmsg) + bmsg_ref[...]
        m2 = jnp.mean(msg2, 1, keepdims=True)
        dv = msg2 - m2
        v2 = jnp.mean(dv * dv, 1, keepdims=True)
        res = dv * lax.rsqrt(v2 + 1e-5) * lnmg_ref[...] + lnmb_ref[...]
        o_ref[sl, :] = res.T


def _edge_call(gdst, gsrc, ef, e0, n_edges,
               We, be, Wm, bm, Wmsg, bmsg, lng, lnb, lnmg, lnmb):
    B = 512
    grid = (n_edges // B,)
    e0b = e0 // B
    return pl.pallas_call(
        _edge_body,
        grid=grid,
        in_specs=[
            pl.BlockSpec((B, 3 * HC), lambda i: (i, 0)),
            pl.BlockSpec((B, 2 * HC), lambda i: (i, 0)),
            pl.BlockSpec((B, C), lambda i: (i + e0b, 0)),
            pl.BlockSpec((C, HC), lambda i: (0, 0)),
            pl.BlockSpec((1, HC), lambda i: (0, 0)),
            pl.BlockSpec((3 * C, 3 * C), lambda i: (0, 0)),
            pl.BlockSpec((1, 3 * C), lambda i: (0, 0)),
            pl.BlockSpec((3 * C, C), lambda i: (0, 0)),
            pl.BlockSpec((1, C), lambda i: (0, 0)),
            pl.BlockSpec((1, 3 * C), lambda i: (0, 0)),
            pl.BlockSpec((1, 3 * C), lambda i: (0, 0)),
            pl.BlockSpec((1, C), lambda i: (0, 0)),
            pl.BlockSpec((1, C), lambda i: (0, 0)),
        ],
        out_specs=pl.BlockSpec((HC, B), lambda i: (0, i)),
        out_shape=jax.ShapeDtypeStruct((HC, n_edges), F32),
    )(gdst, gsrc, ef, We, be.reshape(1, HC), Wm, bm.reshape(1, 3 * C),
      Wmsg, bmsg.reshape(1, C), lng.reshape(1, 3 * C), lnb.reshape(1, 3 * C),
      lnmg.reshape(1, C), lnmb.reshape(1, C))


def _epi_body(*refs):
    n_agg = len(refs) - 8
    (nf_ref, Wc_ref, bc_ref, bng_ref, bnb_ref, Ws_ref, bs_ref, o_ref) = \
        refs[n_agg:]
    agg = refs[0][...]
    for r in refs[1:n_agg]:
        agg = agg + r[...]
    out = lax.dot_general(agg, Wc_ref[...],
                          (((0,), (0,)), ((), ())),
                          preferred_element_type=F32,
                          precision=lax.Precision.HIGHEST) + bc_ref[...]
    mu = jnp.mean(out, 0, keepdims=True)
    dv = out - mu
    var = jnp.mean(dv * dv, 0, keepdims=True)
    out = dv * lax.rsqrt(var + 1e-5) * bng_ref[...] + bnb_ref[...]
    out = out * jax.nn.sigmoid(out)
    o_ref[...] = out + _dot(nf_ref[...], Ws_ref[...]) + bs_ref[...]


def _epi_call(aggs, nf, Wc, bc, bng, bnb, Ws, bs):
    return pl.pallas_call(
        _epi_body,
        out_shape=jax.ShapeDtypeStruct((N_NODES, C), F32),
    )(*aggs, nf, Wc, bc.reshape(1, C), bng.reshape(1, C),
      bnb.reshape(1, C), Ws, bs.reshape(1, C))


def _pool_body(nf_ref, batch_ref, fcW_ref, fcb_ref, oW_ref, ob_ref, o_ref):
    ids = lax.broadcasted_iota(jnp.int32, (N_GRAPHS, 1), 0)
    oh = (batch_ref[...] == ids).astype(F32)
    counts = jnp.sum(oh, 1, keepdims=True)
    pooled = _dot(oh, nf_ref[...]) / jnp.maximum(counts, 1.0)
    feat = _dot(pooled, fcW_ref[...]) + fcb_ref[...]
    feat = feat * jax.nn.sigmoid(feat)
    o_ref[...] = _dot(feat, oW_ref[...]) + ob_ref[...]


def _pool_call(nf, batch_row, fcW, fcb, oW, ob):
    return pl.pallas_call(
        _pool_body,
        out_shape=jax.ShapeDtypeStruct((N_GRAPHS, 1), F32),
    )(nf, batch_row, fcW, fcb.reshape(1, C), oW, ob.reshape(1, 1))


def _sc_gather2(tdst, tsrc, dsti, srci, n_edges):
    epw = n_edges // SC_WORKERS
    gch = 24
    nch = epw // gch

    @functools.partial(
        pl.kernel,
        mesh=plsc.VectorSubcoreMesh(**_SC_MESH),
        out_type=(jax.ShapeDtypeStruct((n_edges, 3 * HC), F32),
                  jax.ShapeDtypeStruct((n_edges, 2 * HC), F32)),
        scratch_types=[
            pltpu.VMEM((gch,), jnp.int32),
            pltpu.VMEM((gch,), jnp.int32),
            pltpu.VMEM((gch, 3 * HC), F32),
            pltpu.VMEM((gch, 3 * HC), F32),
            pltpu.VMEM((gch, 2 * HC), F32),
            pltpu.VMEM((gch, 2 * HC), F32),
            pltpu.SemaphoreType.DMA,
            pltpu.SemaphoreType.DMA,
            pltpu.SemaphoreType.DMA,
            pltpu.SemaphoreType.DMA,
            pltpu.SemaphoreType.DMA,
            pltpu.SemaphoreType.DMA,
        ],
    )
    def k(tdst_hbm, tsrc_hbm, dsti_hbm, srci_hbm, gdst_hbm, gsrc_hbm,
          ib0, ib1, rd0, rd1, rs0, rs1, is0, is1, gs0, gs1, ws0, ws1):
        wid = lax.axis_index("s") * SC_CORES + lax.axis_index("c")
        base = wid * epw

        def run_phase(table, idxh, outh, rb):
            ib = (ib0, ib1)
            isems = (is0, is1)
            gsems = (gs0, gs1)
            wsems = (ws0, ws1)

            def idx_start(i, b):
                pltpu.async_copy(idxh.at[pl.ds(base + i * gch, gch)], ib[b],
                                 isems[b])

            def idx_wait(b):
                pltpu.make_async_copy(idxh.at[pl.ds(0, gch)], ib[b],
                                      isems[b]).wait()

            def g_start(b):
                pltpu.async_copy(table.at[ib[b]], rb[b], gsems[b])

            def g_wait(b):
                pltpu.make_async_copy(table.at[ib[b]], rb[b], gsems[b]).wait()

            def w_start(i, b):
                pltpu.async_copy(rb[b], outh.at[pl.ds(base + i * gch, gch)],
                                 wsems[b])

            def w_wait(b):
                pltpu.make_async_copy(rb[b], outh.at[pl.ds(0, gch)],
                                      wsems[b]).wait()

            idx_start(0, 0)
            idx_wait(0)
            g_start(0)
            idx_start(1, 1)

            def body(i2, carry):
                i = 2 * i2
                g_wait(0)
                w_start(i, 0)
                idx_wait(1)

                @pl.when(i2 > 0)
                def _():
                    w_wait(1)

                g_start(1)

                @pl.when(i + 2 < nch)
                def _():
                    idx_start(i + 2, 0)

                g_wait(1)
                w_start(i + 1, 1)

                @pl.when(i + 2 < nch)
                def _():
                    idx_wait(0)
                    w_wait(0)
                    g_start(0)

                @pl.when(i + 3 < nch)
                def _():
                    idx_start(i + 3, 1)

                return carry

            lax.fori_loop(0, nch // 2, body, 0)
            w_wait(0)
            w_wait(1)

        run_phase(tdst_hbm, dsti_hbm, gdst_hbm, (rd0, rd1))
        run_phase(tsrc_hbm, srci_hbm, gsrc_hbm, (rs0, rs1))

    return k(tdst, tsrc, dsti, srci)


def _sc_scatter_add(msg2_t, idx, zeros, n_edges):
    rows = 16
    sch = 512
    nch = n_edges // sch

    @functools.partial(
        pl.kernel,
        mesh=plsc.VectorSubcoreMesh(**_SC_MESH),
        out_type=jax.ShapeDtypeStruct((HC, N_NODES), F32),
        scratch_types=[
            pltpu.VMEM((sch,), jnp.int32),
            pltpu.VMEM((sch,), jnp.int32),
            pltpu.VMEM((rows, sch), F32),
            pltpu.VMEM((rows, sch), F32),
            pltpu.VMEM((rows, N_NODES), F32),
            pltpu.SemaphoreType.DMA,
            pltpu.SemaphoreType.DMA,
            pltpu.SemaphoreType.DMA,
            pltpu.SemaphoreType.DMA,
        ],
        compiler_params=pltpu.CompilerParams(needs_layout_passes=False),
    )
    def k(msg_hbm, idx_hbm, zeros_hbm, agg_hbm,
          idx0, idx1, rv0, rv1, acc_v, is0, is1, ds0, ds1):
        wid = lax.axis_index("s") * SC_CORES + lax.axis_index("c")
        r0 = wid * rows
        pltpu.sync_copy(zeros_hbm, acc_v)
        lanes = lax.iota(jnp.int32, 16)
        idx_bufs = (idx0, idx1)
        row_bufs = (rv0, rv1)
        isems = (is0, is1)
        dsems = (ds0, ds1)

        def start(i, b):
            off = i * sch
            pltpu.async_copy(idx_hbm.at[pl.ds(off, sch)], idx_bufs[b],
                             isems[b])
            pltpu.async_copy(msg_hbm.at[pl.ds(r0, rows), pl.ds(off, sch)],
                             row_bufs[b], dsems[b])

        def wait(b):
            pltpu.make_async_copy(idx_hbm.at[pl.ds(0, sch)], idx_bufs[b],
                                  isems[b]).wait()
            pltpu.make_async_copy(msg_hbm.at[pl.ds(r0, rows), pl.ds(0, sch)],
                                  row_bufs[b], dsems[b]).wait()

        def compute(b):
            idx_v = idx_bufs[b]
            rows_v = row_bufs[b]

            def group(g, carry2):
                dstv = idx_v[pl.ds(g * 16, 16)]
                for f in range(rows):
                    vals = rows_v[f, pl.ds(g * 16, 16)]
                    plsc.addupdate_scatter(
                        acc_v, [jnp.full((16,), f, jnp.int32), dstv], vals)
                return carry2

            lax.fori_loop(0, sch // 16, group, 0)

        start(0, 0)

        def body2(i2, carry):
            i = i2 * 2

            @pl.when(i + 1 < nch)
            def _():
                start(i + 1, 1)

            wait(0)
            compute(0)

            @pl.when(i + 2 < nch)
            def _():
                start(i + 2, 0)

            @pl.when(i + 1 < nch)
            def _():
                wait(1)
                compute(1)

            return carry

        lax.fori_loop(0, (nch + 1) // 2, body2, 0)
        pltpu.sync_copy(acc_v, agg_hbm.at[pl.ds(r0, rows)])

    return k(msg2_t, idx, zeros)


def kernel(x, edge_index, edge_attr, batch, params):
    p = params
    src = edge_index[0].astype(jnp.int32)
    dst = edge_index[1].astype(jnp.int32)
    batch_row = batch.astype(jnp.int32).reshape(1, N_NODES)
    d2 = jnp.sum(edge_attr * edge_attr, axis=1, keepdims=True)
    zeros = jnp.zeros((16, N_NODES), F32)

    nf = _prep_call(x, p['atom_W'], p['atom_b'])
    ef = _ef_call(d2, p['rbf_W1'], p['rbf_b1'], p['rbf_W2'], p['rbf_b2'])

    NSPLIT = 4
    ES = N_EDGES // NSPLIT
    halves = tuple((dst[i * ES:(i + 1) * ES], src[i * ES:(i + 1) * ES], i * ES)
                   for i in range(NSPLIT))
    for l in range(CONV_LAYERS):
        tdst, tsrc = _qkv_call(nf, p['Wq'][l], p['bq'][l], p['Wk'][l],
                               p['bk'][l], p['Wv'][l], p['bv'][l])
        aggs = []
        gathered = [_sc_gather2(tdst, tsrc, d_h, s_h, ES)
                    for d_h, s_h, _ in halves]
        for (d_h, s_h, e0), (gd, gs) in zip(halves, gathered):
            m_h = _edge_call(gd, gs, ef, e0, ES, p['We'][l], p['be'][l],
                             p['Wm'][l], p['bm'][l], p['Wmsg'][l],
                             p['bmsg'][l], p['ln_g'][l], p['ln_b'][l],
                             p['lnm_g'][l], p['lnm_b'][l])
            aggs.append(_sc_scatter_add(m_h, d_h, zeros, ES))
        nf = _epi_call(aggs, nf, p['Wc'][l], p['bc'][l],
                       p['bn_g'][l], p['bn_b'][l], p['Ws'][l], p['bs'][l])

    out = _pool_call(nf, batch_row, p['fc_W'], p['fc_b'], p['out_W'], p['out_b'])
    return out.reshape(N_GRAPHS)
